# Initial kernel scaffold; baseline (speedup 1.0000x reference)
import jax
import jax.numpy as jnp
from jax.experimental import pallas as pl


def kernel(batch_x, batch_x_mark, batch_y_mark, params):
    raise NotImplementedError("write your pallas kernel here")


# speedup vs baseline: 3.9865x; 3.9865x over previous
import functools
import math

import jax
import jax.numpy as jnp
import numpy as np
from jax.experimental import pallas as pl
from jax.experimental.pallas import tpu as pltpu

B = 4
SEQ = 2048
ENC_IN = 7
C_OUT = 7
MARK_DIM = 4
D_MODEL = 256
N_HEADS = 8
HD = D_MODEL // N_HEADS
D_FF = 1024
FACTOR = 5
LABEL_LEN = 1024
PRED_LEN = 1024

_NEG = -1e30


def _pos_embed_np(L, d):
    pos = np.arange(L)[:, None].astype(np.float64)
    div = np.exp(np.arange(0, d, 2) * -(np.log(10000.0) / d))
    pe = np.zeros((L, d))
    pe[:, 0::2] = np.sin(pos * div)
    pe[:, 1::2] = np.cos(pos * div)
    return pe.astype(np.float32)


_POS_2048 = _pos_embed_np(SEQ, D_MODEL)


def _u_part(Lk):
    return min(FACTOR * int(np.ceil(np.log(Lk))), Lk)


def _cnt_matrix_np(fold, LQ, LK):
    U = _u_part(LK)
    cpu = jax.local_devices(backend="cpu")[0]
    with jax.default_device(cpu):
        key = jax.random.fold_in(jax.random.key(42), fold)
        idx = np.asarray(jax.random.randint(key, (LQ, U), 0, LK))
    cnt = np.zeros((LQ, LK), np.int8)
    np.add.at(cnt, (np.arange(LQ)[:, None], idx), 1)
    return cnt


_CNTS = {
    0: _cnt_matrix_np(0, SEQ, SEQ),
    1: _cnt_matrix_np(1, SEQ // 2, SEQ // 2),
    2: _cnt_matrix_np(2, SEQ, SEQ),
    3: _cnt_matrix_np(3, SEQ, SEQ // 2),
}


def _embed_krn(xc_ref, w_ref, pos_ref, o_ref):
    o_ref[0] = jnp.dot(xc_ref[0], w_ref[...],
                       preferred_element_type=jnp.float32) + pos_ref[...]


def _embed(xc, w, pos):
    b, L, kin = xc.shape
    TL = 256
    return pl.pallas_call(
        _embed_krn,
        grid=(b, L // TL),
        in_specs=[
            pl.BlockSpec((1, TL, kin), lambda i, j: (i, j, 0)),
            pl.BlockSpec((kin, D_MODEL), lambda i, j: (0, 0)),
            pl.BlockSpec((TL, D_MODEL), lambda i, j: (j, 0)),
        ],
        out_specs=pl.BlockSpec((1, TL, D_MODEL), lambda i, j: (i, j, 0)),
        out_shape=jax.ShapeDtypeStruct((b, L, D_MODEL), jnp.float32),
    )(xc, w, pos)


def _qkv_krn(x_ref, w_ref, b_ref, *o_refs):
    y = jnp.dot(x_ref[0], w_ref[...],
                preferred_element_type=jnp.float32) + b_ref[...]
    for t, o_ref in enumerate(o_refs):
        for h in range(N_HEADS):
            c = t * D_MODEL + h * HD
            o_ref[0, h] = y[:, c:c + HD]


def _qkv_proj(x, w, b, n_out):
    b_, L, K = x.shape
    TL = 256
    outs = pl.pallas_call(
        _qkv_krn,
        grid=(b_, L // TL),
        in_specs=[
            pl.BlockSpec((1, TL, K), lambda i, j: (i, j, 0)),
            pl.BlockSpec((K, n_out * D_MODEL), lambda i, j: (0, 0)),
            pl.BlockSpec((1, n_out * D_MODEL), lambda i, j: (0, 0)),
        ],
        out_specs=[pl.BlockSpec((1, N_HEADS, TL, HD), lambda i, j: (i, 0, j, 0))
                   for _ in range(n_out)],
        out_shape=[jax.ShapeDtypeStruct((b_, N_HEADS, L, HD), jnp.float32)
                   for _ in range(n_out)],
    )(x, w, b)
    return outs


def _ln(y, g, b):
    m = jnp.mean(y, axis=-1, keepdims=True)
    v = jnp.mean((y - m) ** 2, axis=-1, keepdims=True)
    return (y - m) * jax.lax.rsqrt(v + 1e-5) * g + b


def _proj_res_ln_krn(x_ref, r_ref, w_ref, b_ref, g_ref, be_ref, o_ref):
    x = jnp.concatenate([x_ref[0, h] for h in range(N_HEADS)], axis=1)
    y = jnp.dot(x, w_ref[...],
                preferred_element_type=jnp.float32) + b_ref[...] + r_ref[...]
    o_ref[...] = _ln(y, g_ref[...], be_ref[...])


def _proj_res_ln(ctx, resid, w, b, g, beta, b_, L):
    N = w.shape[1]
    TL = 256
    nl = L // TL
    return pl.pallas_call(
        _proj_res_ln_krn,
        grid=(b_, nl),
        in_specs=[
            pl.BlockSpec((1, N_HEADS, TL, HD), lambda i, j: (i, 0, j, 0)),
            pl.BlockSpec((TL, N), lambda i, j: (i * nl + j, 0)),
            pl.BlockSpec((D_MODEL, N), lambda i, j: (0, 0)),
            pl.BlockSpec((1, N), lambda i, j: (0, 0)),
            pl.BlockSpec((1, N), lambda i, j: (0, 0)),
            pl.BlockSpec((1, N), lambda i, j: (0, 0)),
        ],
        out_specs=pl.BlockSpec((TL, N), lambda i, j: (i * nl + j, 0)),
        out_shape=jax.ShapeDtypeStruct((b_ * L, N), jnp.float32),
    )(ctx, resid, w, b, g, beta)


def _gelu_exact(x):
    return x * 0.5 * (1.0 + jax.lax.erf(x * (1.0 / math.sqrt(2.0))))


def _ffn_krn(x_ref, w1_ref, b1_ref, w2_ref, b2_ref, g_ref, be_ref,
             g2_ref, be2_ref, o_ref, *, two_ln):
    x = x_ref[...]
    h = _gelu_exact(jnp.dot(x, w1_ref[...],
                            preferred_element_type=jnp.float32) + b1_ref[...])
    y = x + jnp.dot(h, w2_ref[...],
                    preferred_element_type=jnp.float32) + b2_ref[...]
    y = _ln(y, g_ref[...], be_ref[...])
    if two_ln:
        y = _ln(y, g2_ref[...], be2_ref[...])
    o_ref[...] = y


def _ffn(x, w1, b1, w2, b2, g, beta, g2=None, beta2=None):
    M, K = x.shape
    two_ln = g2 is not None
    if g2 is None:
        g2, beta2 = g, beta
    TM = 256
    return pl.pallas_call(
        functools.partial(_ffn_krn, two_ln=two_ln),
        grid=(M // TM,),
        in_specs=[
            pl.BlockSpec((TM, K), lambda i: (i, 0)),
            pl.BlockSpec((K, D_FF), lambda i: (0, 0)),
            pl.BlockSpec((1, D_FF), lambda i: (0, 0)),
            pl.BlockSpec((D_FF, K), lambda i: (0, 0)),
            pl.BlockSpec((1, K), lambda i: (0, 0)),
            pl.BlockSpec((1, K), lambda i: (0, 0)),
            pl.BlockSpec((1, K), lambda i: (0, 0)),
            pl.BlockSpec((1, K), lambda i: (0, 0)),
            pl.BlockSpec((1, K), lambda i: (0, 0)),
        ],
        out_specs=pl.BlockSpec((TM, K), lambda i: (i, 0)),
        out_shape=jax.ShapeDtypeStruct((M, K), jnp.float32),
    )(x, w1, b1, w2, b2, g, beta, g2, beta2)


def _measure_krn(q_ref, k_ref, c_ref, m_ref, *, LK):
    q = q_ref[0, 0]
    k = k_ref[0, 0]
    s = jax.lax.dot_general(q, k, (((1,), (1,)), ((), ())),
                            preferred_element_type=jnp.float32)
    cf = c_ref[...].astype(jnp.float32)
    mx = jnp.max(jnp.where(cf > 0.5, s, _NEG), axis=-1)
    sm = jnp.sum(s * cf, axis=-1) * (1.0 / LK)
    m_ref[0, 0, :] = mx - sm


def _measure(q, k, cnt):
    b, _, LQ, _ = q.shape
    LK = k.shape[2]
    TQ = 256
    m = pl.pallas_call(
        functools.partial(_measure_krn, LK=LK),
        grid=(b * N_HEADS, LQ // TQ),
        in_specs=[
            pl.BlockSpec((1, 1, TQ, HD),
                         lambda g, i: (g // N_HEADS, g % N_HEADS, i, 0)),
            pl.BlockSpec((1, 1, LK, HD),
                         lambda g, i: (g // N_HEADS, g % N_HEADS, 0, 0)),
            pl.BlockSpec((TQ, LK), lambda g, i: (i, 0)),
        ],
        out_specs=pl.BlockSpec((1, 1, TQ), lambda g, i: (g, 0, i)),
        out_shape=jax.ShapeDtypeStruct((b * N_HEADS, 1, LQ), jnp.float32),
    )(q, k, cnt)
    return m.reshape(b * N_HEADS, LQ)


def _topk_krn(m_ref, o_ref, *, u, LQ, BH):
    m = m_ref[...]
    lane = jax.lax.broadcasted_iota(jnp.int32, (BH, LQ), 1)
    lane128 = jax.lax.broadcasted_iota(jnp.int32, (BH, 128), 1)
    acc = jnp.zeros((BH, 128), jnp.int32)
    for i in range(u):
        mx = jnp.max(m, axis=-1, keepdims=True)
        am = jnp.min(jnp.where(m == mx, lane, LQ), axis=-1, keepdims=True)
        acc = jnp.where(lane128 == i, am, acc)
        m = jnp.where(lane == am, _NEG, m)
    o_ref[...] = acc


def _topk(m, u):
    BH, LQ = m.shape
    return pl.pallas_call(
        functools.partial(_topk_krn, u=u, LQ=LQ, BH=BH),
        out_shape=jax.ShapeDtypeStruct((BH, 128), jnp.int32),
    )(m)


def _sattn_krn(idx_ref, q_ref, k_ref, v_ref, o_ref, *, u, LQ, LK, masked):
    bh = pl.program_id(0)
    V = v_ref[0, 0]
    K = k_ref[0, 0]

    if masked:
        TT = 256
        r_io = jax.lax.broadcasted_iota(jnp.int32, (TT, TT), 0)
        c_io = jax.lax.broadcasted_iota(jnp.int32, (TT, TT), 1)
        tri = (r_io >= c_io).astype(jnp.float32)
        carry = jnp.zeros((1, HD), jnp.float32)
        for t in range(LK // TT):
            vt = V[t * TT:(t + 1) * TT, :]
            cum = jnp.dot(tri, vt, preferred_element_type=jnp.float32) + carry
            o_ref[0, 0, t * TT:(t + 1) * TT, :] = cum
            carry = cum[TT - 1:TT, :]
    else:
        mean = jnp.sum(V, axis=0, keepdims=True) * (1.0 / LK)
        o_ref[0, 0] = jnp.broadcast_to(mean, (LQ, HD))

    rows = []
    scalars = []
    for i in range(u):
        r = idx_ref[bh, i]
        scalars.append(r)
        rows.append(q_ref[0, 0, pl.ds(r, 1), :])
    upad = ((u + 7) // 8) * 8
    if upad > u:
        rows.append(jnp.zeros((upad - u, HD), jnp.float32))
    qred = jnp.concatenate(rows, axis=0)

    s = jax.lax.dot_general(qred, K, (((1,), (1,)), ((), ())),
                            preferred_element_type=jnp.float32)
    s = s * (1.0 / math.sqrt(HD))
    if masked:
        col = jnp.concatenate(
            [jnp.full((1, 1), r, jnp.int32) for r in scalars]
            + ([jnp.full((upad - u, 1), LK, jnp.int32)] if upad > u else []),
            axis=0)
        lane = jax.lax.broadcasted_iota(jnp.int32, (upad, LK), 1)
        s = jnp.where(lane > col, _NEG, s)
    mx = jnp.max(s, axis=-1, keepdims=True)
    e = jnp.exp(s - mx)
    p = e / jnp.sum(e, axis=-1, keepdims=True)
    upd = jnp.dot(p, V, preferred_element_type=jnp.float32)

    for i in range(u):
        o_ref[0, 0, pl.ds(scalars[i], 1), :] = upd[i:i + 1, :]


def _sparse_attn(idx, q, k, v, u, masked):
    b, _, LQ, _ = q.shape
    LK = k.shape[2]
    return pl.pallas_call(
        functools.partial(_sattn_krn, u=u, LQ=LQ, LK=LK, masked=masked),
        grid=(b * N_HEADS,),
        in_specs=[
            pl.BlockSpec(memory_space=pltpu.SMEM),
            pl.BlockSpec((1, 1, LQ, HD),
                         lambda g: (g // N_HEADS, g % N_HEADS, 0, 0)),
            pl.BlockSpec((1, 1, LK, HD),
                         lambda g: (g // N_HEADS, g % N_HEADS, 0, 0)),
            pl.BlockSpec((1, 1, LK, HD),
                         lambda g: (g // N_HEADS, g % N_HEADS, 0, 0)),
        ],
        out_specs=pl.BlockSpec((1, 1, LQ, HD),
                               lambda g: (g // N_HEADS, g % N_HEADS, 0, 0)),
        out_shape=jax.ShapeDtypeStruct((b, N_HEADS, LQ, HD), jnp.float32),
    )(idx, q, k, v)


def _conv_distil_krn(x_ref, w0_ref, w1_ref, w2_ref, b_ref, sc_ref, bb_ref,
                     o_ref, *, L):
    x = x_ref[0]
    xm1 = jnp.concatenate([x[L - 1:L, :], x[:L - 1, :]], axis=0)
    xp1 = jnp.concatenate([x[1:, :], x[:1, :]], axis=0)
    y = (jnp.dot(xm1, w0_ref[...], preferred_element_type=jnp.float32)
         + jnp.dot(x, w1_ref[...], preferred_element_type=jnp.float32)
         + jnp.dot(xp1, w2_ref[...], preferred_element_type=jnp.float32)
         + b_ref[...])
    y = y * sc_ref[...] + bb_ref[...]
    y = jnp.where(y > 0, y, jnp.exp(jnp.minimum(y, 0.0)) - 1.0)
    C = y.shape[1]
    neg = jnp.full((1, C), _NEG, jnp.float32)
    ym1 = jnp.concatenate([neg, y[:L - 1, :]], axis=0)
    yp1 = jnp.concatenate([y[1:, :], neg], axis=0)
    pf = jnp.maximum(jnp.maximum(ym1, y), yp1)
    o_ref[0] = pf


def _conv_distil(x, w0, w1, w2, b, scale, bb):
    b_, L, C = x.shape
    full = pl.pallas_call(
        functools.partial(_conv_distil_krn, L=L),
        grid=(b_,),
        in_specs=[
            pl.BlockSpec((1, L, C), lambda i: (i, 0, 0)),
            pl.BlockSpec((C, C), lambda i: (0, 0)),
            pl.BlockSpec((C, C), lambda i: (0, 0)),
            pl.BlockSpec((C, C), lambda i: (0, 0)),
            pl.BlockSpec((1, C), lambda i: (0, 0)),
            pl.BlockSpec((1, C), lambda i: (0, 0)),
            pl.BlockSpec((1, C), lambda i: (0, 0)),
        ],
        out_specs=pl.BlockSpec((1, L, C), lambda i: (i, 0, 0)),
        out_shape=jax.ShapeDtypeStruct((b_, L, C), jnp.float32),
    )(x, w0, w1, w2, b, scale, bb)
    return full[:, ::2, :]


def _final_krn(x_ref, g_ref, be_ref, w_ref, b_ref, o_ref):
    y = _ln(x_ref[...], g_ref[...], be_ref[...])
    o_ref[...] = jnp.dot(y, w_ref[...],
                         preferred_element_type=jnp.float32) + b_ref[...]


def _final_proj(x, g, beta, w, b):
    M, K = x.shape
    N = w.shape[1]
    TM = 256
    return pl.pallas_call(
        _final_krn,
        grid=(M // TM,),
        in_specs=[
            pl.BlockSpec((TM, K), lambda i: (i, 0)),
            pl.BlockSpec((1, K), lambda i: (0, 0)),
            pl.BlockSpec((1, K), lambda i: (0, 0)),
            pl.BlockSpec((K, N), lambda i: (0, 0)),
            pl.BlockSpec((1, N), lambda i: (0, 0)),
        ],
        out_specs=pl.BlockSpec((TM, N), lambda i: (i, 0)),
        out_shape=jax.ShapeDtypeStruct((M, N), jnp.float32),
    )(x, g, beta, w, b)


def _row(v):
    return v.reshape(1, -1)


def _attn_block(p, xq, xkv, fold, masked, ln_g, ln_b):
    b, LQ, _ = xq.shape
    LK = xkv.shape[1]
    xq2 = xq.reshape(b * LQ, D_MODEL)
    if xq is xkv:
        wqkv = jnp.concatenate([p['wq'].T, p['wk'].T, p['wv'].T], axis=1)
        bqkv = jnp.concatenate([p['bq'], p['bk'], p['bv']])
        q, k, v = _qkv_proj(xq, wqkv, _row(bqkv), 3)
    else:
        q, = _qkv_proj(xq, p['wq'].T, _row(p['bq']), 1)
        wkv = jnp.concatenate([p['wk'].T, p['wv'].T], axis=1)
        bkv = jnp.concatenate([p['bk'], p['bv']])
        k, v = _qkv_proj(xkv, wkv, _row(bkv), 2)

    cnt = jnp.asarray(_CNTS[fold])
    u = min(FACTOR * int(np.ceil(np.log(LQ))), LQ)
    m = _measure(q, k, cnt)
    idx = _topk(m, u)
    ctx = _sparse_attn(idx, q, k, v, u, masked)
    y = _proj_res_ln(ctx, xq2, p['wo'].T, _row(p['bo']),
                     _row(ln_g), _row(ln_b), b, LQ)
    return y.reshape(b, LQ, D_MODEL)


def _embed_inputs(x, mark, conv_w, temp_w):
    b, L, _ = x.shape
    xc = jnp.concatenate(
        [jnp.roll(x, 1, axis=1), x, jnp.roll(x, -1, axis=1), mark], axis=-1)
    kin = xc.shape[-1]
    pad = (-kin) % 32
    xc = jnp.pad(xc, ((0, 0), (0, 0), (0, pad)))
    w = jnp.concatenate([conv_w[:, :, 0].T, conv_w[:, :, 1].T,
                         conv_w[:, :, 2].T, temp_w.T], axis=0)
    w = jnp.pad(w, ((0, pad), (0, 0)))
    return _embed(xc, w, jnp.asarray(_POS_2048[:L]))


def kernel(batch_x, batch_x_mark, batch_y_mark, params):
    p = params

    enc = _embed_inputs(batch_x, batch_x_mark,
                        p['enc_emb']['conv_w'], p['enc_emb']['temp_w'])

    e0 = p['enc0']
    x = _attn_block(e0['attn'], enc, enc, 0, False, e0['ln1_g'], e0['ln1_b'])
    x2 = _ffn(x.reshape(-1, D_MODEL), e0['w1'].T, _row(e0['b1']),
              e0['w2'].T, _row(e0['b2']), _row(e0['ln2_g']), _row(e0['ln2_b']))
    x = x2.reshape(B, SEQ, D_MODEL)

    c0 = p['conv0']
    scale = _row(c0['bn_g'] * (1.0 / np.sqrt(1.0 + 1e-5)))
    x = _conv_distil(x, c0['w'][:, :, 0].T, c0['w'][:, :, 1].T,
                     c0['w'][:, :, 2].T, _row(c0['b']), scale, _row(c0['bn_b']))

    e1 = p['enc1']
    x = _attn_block(e1['attn'], x, x, 1, False, e1['ln1_g'], e1['ln1_b'])
    x2 = _ffn(x.reshape(-1, D_MODEL), e1['w1'].T, _row(e1['b1']),
              e1['w2'].T, _row(e1['b2']), _row(e1['ln2_g']), _row(e1['ln2_b']),
              _row(p['enc_norm_g']), _row(p['enc_norm_b']))
    enc_out = x2.reshape(B, SEQ // 2, D_MODEL)

    dec_inp = jnp.concatenate(
        [batch_x[:, -LABEL_LEN:, :],
         jnp.zeros((B, PRED_LEN, ENC_IN), jnp.float32)], axis=1)
    dec = _embed_inputs(dec_inp, batch_y_mark,
                        p['dec_emb']['conv_w'], p['dec_emb']['temp_w'])

    d0 = p['dec0']
    L_DEC = LABEL_LEN + PRED_LEN
    x = _attn_block(d0['self'], dec, dec, 2, True, d0['ln1_g'], d0['ln1_b'])
    x = _attn_block(d0['cross'], x, enc_out, 3, False, d0['ln2_g'], d0['ln2_b'])
    x2 = _ffn(x.reshape(-1, D_MODEL), d0['w1'].T, _row(d0['b1']),
              d0['w2'].T, _row(d0['b2']), _row(d0['ln3_g']), _row(d0['ln3_b']))

    xdec = x2.reshape(B, L_DEC, D_MODEL)[:, -PRED_LEN:, :]
    out = _final_proj(xdec.reshape(B * PRED_LEN, D_MODEL),
                      _row(p['dec_norm_g']), _row(p['dec_norm_b']),
                      p['proj_w'].T, _row(p['proj_b']))
    return out.reshape(B, PRED_LEN, C_OUT)


# speedup vs baseline: 4.9947x; 1.2529x over previous
import functools
import math

import jax
import jax.numpy as jnp
import numpy as np
from jax.experimental import pallas as pl
from jax.experimental.pallas import tpu as pltpu

B = 4
SEQ = 2048
ENC_IN = 7
C_OUT = 7
MARK_DIM = 4
D_MODEL = 256
N_HEADS = 8
HD = D_MODEL // N_HEADS
D_FF = 1024
FACTOR = 5
LABEL_LEN = 1024
PRED_LEN = 1024

_NEG = -1e30


def _pos_embed_np(L, d):
    pos = np.arange(L)[:, None].astype(np.float64)
    div = np.exp(np.arange(0, d, 2) * -(np.log(10000.0) / d))
    pe = np.zeros((L, d))
    pe[:, 0::2] = np.sin(pos * div)
    pe[:, 1::2] = np.cos(pos * div)
    return pe.astype(np.float32)


_POS_2048 = _pos_embed_np(SEQ, D_MODEL)


def _u_part(Lk):
    return min(FACTOR * int(np.ceil(np.log(Lk))), Lk)


def _tf_rotl(x, d):
    return ((x << np.uint32(d)) | (x >> np.uint32(32 - d))).astype(np.uint32)


def _tf2x32(k0, k1, x0, x1):
    x0 = x0.astype(np.uint32).copy()
    x1 = x1.astype(np.uint32).copy()
    ks0, ks1 = np.uint32(k0), np.uint32(k1)
    ks2 = np.uint32(ks0 ^ ks1 ^ np.uint32(0x1BD11BDA))
    rot = [13, 15, 26, 6, 17, 29, 16, 24]
    x0 = (x0 + ks0).astype(np.uint32)
    x1 = (x1 + ks1).astype(np.uint32)
    keys = [(ks1, ks2), (ks2, ks0), (ks0, ks1), (ks1, ks2), (ks2, ks0)]
    for r in range(5):
        for d in (rot[:4] if r % 2 == 0 else rot[4:]):
            x0 = (x0 + x1).astype(np.uint32)
            x1 = _tf_rotl(x1, d)
            x1 = (x1 ^ x0).astype(np.uint32)
        ka, kb = keys[r]
        x0 = (x0 + ka).astype(np.uint32)
        x1 = (x1 + kb + np.uint32(r + 1)).astype(np.uint32)
    return x0, x1


def _tf_counter(n):
    cnt = np.arange(n, dtype=np.uint64)
    return ((cnt >> np.uint64(32)).astype(np.uint32),
            (cnt & np.uint64(0xFFFFFFFF)).astype(np.uint32))


def _tf_key(seed):
    return np.uint32(np.uint64(seed) >> np.uint64(32)), np.uint32(seed & 0xFFFFFFFF)


def _tf_fold_in(key, data):
    d0, d1 = _tf_key(int(data))
    x0, x1 = _tf2x32(key[0], key[1], np.array([d0]), np.array([d1]))
    return np.uint32(x0[0]), np.uint32(x1[0])


def _tf_randint_pow2(key, shape, span):
    c0, c1 = _tf_counter(2)
    s0, s1 = _tf2x32(key[0], key[1], c0, c1)
    k2 = (s0[1], s1[1])
    n = int(np.prod(shape))
    c0, c1 = _tf_counter(n)
    b0, b1 = _tf2x32(k2[0], k2[1], c0, c1)
    return ((b0 ^ b1) % np.uint32(span)).astype(np.int64).reshape(shape)


def _cnt_matrix_np(fold, LQ, LK):
    U = _u_part(LK)
    key = _tf_fold_in(_tf_key(42), fold)
    idx = _tf_randint_pow2(key, (LQ, U), LK)
    cnt = np.zeros((LQ, LK), np.int8)
    np.add.at(cnt, (np.arange(LQ)[:, None], idx), 1)
    return cnt


_CNTS_T = {
    0: np.ascontiguousarray(_cnt_matrix_np(0, SEQ, SEQ).T),
    1: np.ascontiguousarray(_cnt_matrix_np(1, SEQ // 2, SEQ // 2).T),
    2: np.ascontiguousarray(_cnt_matrix_np(2, SEQ, SEQ).T),
    3: np.ascontiguousarray(_cnt_matrix_np(3, SEQ, SEQ // 2).T),
}


def _embed_krn(xc_ref, w_ref, pos_ref, o_ref):
    o_ref[0] = jnp.dot(xc_ref[0], w_ref[...],
                       preferred_element_type=jnp.float32) + pos_ref[...]


def _embed(xc, w, pos):
    b, L, kin = xc.shape
    TL = 256
    return pl.pallas_call(
        _embed_krn,
        grid=(b, L // TL),
        in_specs=[
            pl.BlockSpec((1, TL, kin), lambda i, j: (i, j, 0)),
            pl.BlockSpec((kin, D_MODEL), lambda i, j: (0, 0)),
            pl.BlockSpec((TL, D_MODEL), lambda i, j: (j, 0)),
        ],
        out_specs=pl.BlockSpec((1, TL, D_MODEL), lambda i, j: (i, j, 0)),
        out_shape=jax.ShapeDtypeStruct((b, L, D_MODEL), jnp.float32),
    )(xc, w, pos)


def _qkv_krn(x_ref, w_ref, b_ref, *o_refs):
    y = jnp.dot(x_ref[0], w_ref[...],
                preferred_element_type=jnp.float32) + b_ref[...]
    for t, o_ref in enumerate(o_refs):
        for h in range(N_HEADS):
            c = t * D_MODEL + h * HD
            o_ref[0, h] = y[:, c:c + HD]


def _qkv_proj(x, w, b, n_out):
    b_, L, K = x.shape
    TL = 256
    outs = pl.pallas_call(
        _qkv_krn,
        grid=(b_, L // TL),
        in_specs=[
            pl.BlockSpec((1, TL, K), lambda i, j: (i, j, 0)),
            pl.BlockSpec((K, n_out * D_MODEL), lambda i, j: (0, 0)),
            pl.BlockSpec((1, n_out * D_MODEL), lambda i, j: (0, 0)),
        ],
        out_specs=[pl.BlockSpec((1, N_HEADS, TL, HD), lambda i, j: (i, 0, j, 0))
                   for _ in range(n_out)],
        out_shape=[jax.ShapeDtypeStruct((b_, N_HEADS, L, HD), jnp.float32)
                   for _ in range(n_out)],
    )(x, w, b)
    return outs


def _ln(y, g, b):
    m = jnp.mean(y, axis=-1, keepdims=True)
    v = jnp.mean((y - m) ** 2, axis=-1, keepdims=True)
    return (y - m) * jax.lax.rsqrt(v + 1e-5) * g + b


def _proj_res_ln_krn(x_ref, r_ref, w_ref, b_ref, g_ref, be_ref, o_ref):
    x = jnp.concatenate([x_ref[0, h] for h in range(N_HEADS)], axis=1)
    y = jnp.dot(x, w_ref[...],
                preferred_element_type=jnp.float32) + b_ref[...] + r_ref[...]
    o_ref[...] = _ln(y, g_ref[...], be_ref[...])


def _proj_res_ln(ctx, resid, w, b, g, beta, b_, L):
    N = w.shape[1]
    TL = 256
    nl = L // TL
    return pl.pallas_call(
        _proj_res_ln_krn,
        grid=(b_, nl),
        in_specs=[
            pl.BlockSpec((1, N_HEADS, TL, HD), lambda i, j: (i, 0, j, 0)),
            pl.BlockSpec((TL, N), lambda i, j: (i * nl + j, 0)),
            pl.BlockSpec((D_MODEL, N), lambda i, j: (0, 0)),
            pl.BlockSpec((1, N), lambda i, j: (0, 0)),
            pl.BlockSpec((1, N), lambda i, j: (0, 0)),
            pl.BlockSpec((1, N), lambda i, j: (0, 0)),
        ],
        out_specs=pl.BlockSpec((TL, N), lambda i, j: (i * nl + j, 0)),
        out_shape=jax.ShapeDtypeStruct((b_ * L, N), jnp.float32),
    )(ctx, resid, w, b, g, beta)


def _gelu_exact(x):
    return x * 0.5 * (1.0 + jax.lax.erf(x * (1.0 / math.sqrt(2.0))))


def _ffn_krn(x_ref, w1_ref, b1_ref, w2_ref, b2_ref, g_ref, be_ref,
             g2_ref, be2_ref, o_ref, *, two_ln):
    x = x_ref[...]
    h = _gelu_exact(jnp.dot(x, w1_ref[...],
                            preferred_element_type=jnp.float32) + b1_ref[...])
    y = x + jnp.dot(h, w2_ref[...],
                    preferred_element_type=jnp.float32) + b2_ref[...]
    y = _ln(y, g_ref[...], be_ref[...])
    if two_ln:
        y = _ln(y, g2_ref[...], be2_ref[...])
    o_ref[...] = y


def _ffn(x, w1, b1, w2, b2, g, beta, g2=None, beta2=None):
    M, K = x.shape
    two_ln = g2 is not None
    if g2 is None:
        g2, beta2 = g, beta
    TM = 256
    return pl.pallas_call(
        functools.partial(_ffn_krn, two_ln=two_ln),
        grid=(M // TM,),
        in_specs=[
            pl.BlockSpec((TM, K), lambda i: (i, 0)),
            pl.BlockSpec((K, D_FF), lambda i: (0, 0)),
            pl.BlockSpec((1, D_FF), lambda i: (0, 0)),
            pl.BlockSpec((D_FF, K), lambda i: (0, 0)),
            pl.BlockSpec((1, K), lambda i: (0, 0)),
            pl.BlockSpec((1, K), lambda i: (0, 0)),
            pl.BlockSpec((1, K), lambda i: (0, 0)),
            pl.BlockSpec((1, K), lambda i: (0, 0)),
            pl.BlockSpec((1, K), lambda i: (0, 0)),
        ],
        out_specs=pl.BlockSpec((TM, K), lambda i: (i, 0)),
        out_shape=jax.ShapeDtypeStruct((M, K), jnp.float32),
    )(x, w1, b1, w2, b2, g, beta, g2, beta2)


def _fattn_krn(q_ref, k_ref, v_ref, ct_ref, o_ref, m_ref, p_ref, cm_ref,
               *, u, upad, LQ, LK, masked, HG):
    TQ = 256

    for t in range(LQ // TQ):
        cf = ct_ref[:, t * TQ:(t + 1) * TQ].astype(jnp.float32)
        madd = jnp.where(cf > 0.5, 0.0, _NEG)
        for h in range(HG):
            qt = q_ref[0, h, t * TQ:(t + 1) * TQ, :]
            st = jax.lax.dot_general(k_ref[0, h], qt, (((1,), (1,)), ((), ())),
                                     preferred_element_type=jnp.float32)
            mx = jnp.max(st + madd, axis=0, keepdims=True)
            sm = jnp.sum(st * cf, axis=0, keepdims=True) * (1.0 / LK)
            m_ref[h:h + 1, t * TQ:(t + 1) * TQ] = mx - sm

    m = m_ref[...]
    lane_q = jax.lax.broadcasted_iota(jnp.int32, (HG, LQ), 1)
    if masked:
        lane_k = jax.lax.broadcasted_iota(jnp.int32, (HG, LK), 1)
    for i in range(u):
        mx = jnp.max(m, axis=-1, keepdims=True)
        am = jnp.min(jnp.where(m == mx, lane_q, LQ), axis=-1, keepdims=True)
        p_ref[:, i, :] = (lane_q == am).astype(jnp.float32)
        if masked:
            cm_ref[:, i, :] = (lane_k > am).astype(jnp.float32)
        m = jnp.where(lane_q == am, _NEG, m)
    for i in range(u, upad):
        p_ref[:, i, :] = jnp.zeros((HG, LQ), jnp.float32)
        if masked:
            cm_ref[:, i, :] = jnp.zeros((HG, LK), jnp.float32)

    ones = jnp.ones((upad, HD), jnp.float32)
    if masked:
        TT = 256
        r_io = jax.lax.broadcasted_iota(jnp.int32, (TT, TT), 0)
        c_io = jax.lax.broadcasted_iota(jnp.int32, (TT, TT), 1)
        tri = (r_io >= c_io).astype(jnp.float32)
    for h in range(HG):
        V = v_ref[0, h]
        ph = p_ref[h]
        qred = jnp.dot(ph, q_ref[0, h], preferred_element_type=jnp.float32)
        s = jax.lax.dot_general(qred, k_ref[0, h], (((1,), (1,)), ((), ())),
                                preferred_element_type=jnp.float32)
        s = s * (1.0 / math.sqrt(HD))
        if masked:
            s = s + cm_ref[h] * _NEG
        mx = jnp.max(s, axis=-1, keepdims=True)
        e = jnp.exp(s - mx)
        p = e / jnp.sum(e, axis=-1, keepdims=True)
        upd = jnp.dot(p, V, preferred_element_type=jnp.float32)
        ctxsel = jax.lax.dot_general(ph, upd, (((0,), (0,)), ((), ())),
                                     preferred_element_type=jnp.float32)
        covm = jax.lax.dot_general(ph, ones, (((0,), (0,)), ((), ())),
                                   preferred_element_type=jnp.float32)
        if masked:
            carry = jnp.zeros((1, HD), jnp.float32)
            for t in range(LK // TT):
                vt = V[t * TT:(t + 1) * TT, :]
                cum = jnp.dot(tri, vt,
                              preferred_element_type=jnp.float32) + carry
                sl = slice(t * TT, (t + 1) * TT)
                o_ref[0, h, sl, :] = (cum * (1.0 - covm[sl, :])
                                      + ctxsel[sl, :])
                carry = cum[TT - 1:TT, :]
        else:
            mean = jnp.sum(V, axis=0, keepdims=True) * (1.0 / LK)
            base = jnp.broadcast_to(mean, (LQ, HD))
            o_ref[0, h] = base * (1.0 - covm) + ctxsel


def _fused_attn(q, k, v, cntT, u, masked):
    b, _, LQ, _ = q.shape
    LK = k.shape[2]
    upad = ((u + 7) // 8) * 8
    HG = 4
    scratch = [
        pltpu.VMEM((HG, LQ), jnp.float32),
        pltpu.VMEM((HG, upad, LQ), jnp.float32),
        pltpu.VMEM((HG, upad, LK), jnp.float32),
    ]
    return pl.pallas_call(
        functools.partial(_fattn_krn, u=u, upad=upad, LQ=LQ, LK=LK,
                          masked=masked, HG=HG),
        grid=(b, N_HEADS // HG),
        in_specs=[
            pl.BlockSpec((1, HG, LQ, HD), lambda g, j: (g, j, 0, 0)),
            pl.BlockSpec((1, HG, LK, HD), lambda g, j: (g, j, 0, 0)),
            pl.BlockSpec((1, HG, LK, HD), lambda g, j: (g, j, 0, 0)),
            pl.BlockSpec((LK, LQ), lambda g, j: (0, 0)),
        ],
        out_specs=pl.BlockSpec((1, HG, LQ, HD), lambda g, j: (g, j, 0, 0)),
        out_shape=jax.ShapeDtypeStruct((b, N_HEADS, LQ, HD), jnp.float32),
        scratch_shapes=scratch,
    )(q, k, v, cntT)


def _conv_distil_krn(x_ref, w0_ref, w1_ref, w2_ref, b_ref, sc_ref, bb_ref,
                     o_ref, *, L):
    x = x_ref[0]
    xm1 = jnp.concatenate([x[L - 1:L, :], x[:L - 1, :]], axis=0)
    xp1 = jnp.concatenate([x[1:, :], x[:1, :]], axis=0)
    y = (jnp.dot(xm1, w0_ref[...], preferred_element_type=jnp.float32)
         + jnp.dot(x, w1_ref[...], preferred_element_type=jnp.float32)
         + jnp.dot(xp1, w2_ref[...], preferred_element_type=jnp.float32)
         + b_ref[...])
    y = y * sc_ref[...] + bb_ref[...]
    y = jnp.where(y > 0, y, jnp.exp(jnp.minimum(y, 0.0)) - 1.0)
    C = y.shape[1]
    neg = jnp.full((1, C), _NEG, jnp.float32)
    ym1 = jnp.concatenate([neg, y[:L - 1, :]], axis=0)
    yp1 = jnp.concatenate([y[1:, :], neg], axis=0)
    pf = jnp.maximum(jnp.maximum(ym1, y), yp1)
    o_ref[0] = pf


def _conv_distil(x, w0, w1, w2, b, scale, bb):
    b_, L, C = x.shape
    full = pl.pallas_call(
        functools.partial(_conv_distil_krn, L=L),
        grid=(b_,),
        in_specs=[
            pl.BlockSpec((1, L, C), lambda i: (i, 0, 0)),
            pl.BlockSpec((C, C), lambda i: (0, 0)),
            pl.BlockSpec((C, C), lambda i: (0, 0)),
            pl.BlockSpec((C, C), lambda i: (0, 0)),
            pl.BlockSpec((1, C), lambda i: (0, 0)),
            pl.BlockSpec((1, C), lambda i: (0, 0)),
            pl.BlockSpec((1, C), lambda i: (0, 0)),
        ],
        out_specs=pl.BlockSpec((1, L, C), lambda i: (i, 0, 0)),
        out_shape=jax.ShapeDtypeStruct((b_, L, C), jnp.float32),
    )(x, w0, w1, w2, b, scale, bb)
    return full[:, ::2, :]


def _final_krn(x_ref, g_ref, be_ref, w_ref, b_ref, o_ref):
    y = _ln(x_ref[...], g_ref[...], be_ref[...])
    o_ref[...] = jnp.dot(y, w_ref[...],
                         preferred_element_type=jnp.float32) + b_ref[...]


def _final_proj(x, g, beta, w, b):
    M, K = x.shape
    N = w.shape[1]
    TM = 256
    return pl.pallas_call(
        _final_krn,
        grid=(M // TM,),
        in_specs=[
            pl.BlockSpec((TM, K), lambda i: (i, 0)),
            pl.BlockSpec((1, K), lambda i: (0, 0)),
            pl.BlockSpec((1, K), lambda i: (0, 0)),
            pl.BlockSpec((K, N), lambda i: (0, 0)),
            pl.BlockSpec((1, N), lambda i: (0, 0)),
        ],
        out_specs=pl.BlockSpec((TM, N), lambda i: (i, 0)),
        out_shape=jax.ShapeDtypeStruct((M, N), jnp.float32),
    )(x, g, beta, w, b)


def _row(v):
    return v.reshape(1, -1)


def _attn_block(p, xq, xkv, fold, masked, ln_g, ln_b):
    b, LQ, _ = xq.shape
    LK = xkv.shape[1]
    xq2 = xq.reshape(b * LQ, D_MODEL)
    if xq is xkv:
        wqkv = jnp.concatenate([p['wq'].T, p['wk'].T, p['wv'].T], axis=1)
        bqkv = jnp.concatenate([p['bq'], p['bk'], p['bv']])
        q, k, v = _qkv_proj(xq, wqkv, _row(bqkv), 3)
    else:
        q, = _qkv_proj(xq, p['wq'].T, _row(p['bq']), 1)
        wkv = jnp.concatenate([p['wk'].T, p['wv'].T], axis=1)
        bkv = jnp.concatenate([p['bk'], p['bv']])
        k, v = _qkv_proj(xkv, wkv, _row(bkv), 2)

    cntT = jnp.asarray(_CNTS_T[fold])
    u = min(FACTOR * int(np.ceil(np.log(LQ))), LQ)
    ctx = _fused_attn(q, k, v, cntT, u, masked)
    y = _proj_res_ln(ctx, xq2, p['wo'].T, _row(p['bo']),
                     _row(ln_g), _row(ln_b), b, LQ)
    return y.reshape(b, LQ, D_MODEL)


def _embed_inputs(x, mark, conv_w, temp_w):
    b, L, _ = x.shape
    xc = jnp.concatenate(
        [jnp.roll(x, 1, axis=1), x, jnp.roll(x, -1, axis=1), mark], axis=-1)
    kin = xc.shape[-1]
    pad = (-kin) % 32
    xc = jnp.pad(xc, ((0, 0), (0, 0), (0, pad)))
    w = jnp.concatenate([conv_w[:, :, 0].T, conv_w[:, :, 1].T,
                         conv_w[:, :, 2].T, temp_w.T], axis=0)
    w = jnp.pad(w, ((0, pad), (0, 0)))
    return _embed(xc, w, jnp.asarray(_POS_2048[:L]))


def kernel(batch_x, batch_x_mark, batch_y_mark, params):
    p = params

    enc = _embed_inputs(batch_x, batch_x_mark,
                        p['enc_emb']['conv_w'], p['enc_emb']['temp_w'])

    e0 = p['enc0']
    x = _attn_block(e0['attn'], enc, enc, 0, False, e0['ln1_g'], e0['ln1_b'])
    x2 = _ffn(x.reshape(-1, D_MODEL), e0['w1'].T, _row(e0['b1']),
              e0['w2'].T, _row(e0['b2']), _row(e0['ln2_g']), _row(e0['ln2_b']))
    x = x2.reshape(B, SEQ, D_MODEL)

    c0 = p['conv0']
    scale = _row(c0['bn_g'] * (1.0 / np.sqrt(1.0 + 1e-5)))
    x = _conv_distil(x, c0['w'][:, :, 0].T, c0['w'][:, :, 1].T,
                     c0['w'][:, :, 2].T, _row(c0['b']), scale, _row(c0['bn_b']))

    e1 = p['enc1']
    x = _attn_block(e1['attn'], x, x, 1, False, e1['ln1_g'], e1['ln1_b'])
    x2 = _ffn(x.reshape(-1, D_MODEL), e1['w1'].T, _row(e1['b1']),
              e1['w2'].T, _row(e1['b2']), _row(e1['ln2_g']), _row(e1['ln2_b']),
              _row(p['enc_norm_g']), _row(p['enc_norm_b']))
    enc_out = x2.reshape(B, SEQ // 2, D_MODEL)

    dec_inp = jnp.concatenate(
        [batch_x[:, -LABEL_LEN:, :],
         jnp.zeros((B, PRED_LEN, ENC_IN), jnp.float32)], axis=1)
    dec = _embed_inputs(dec_inp, batch_y_mark,
                        p['dec_emb']['conv_w'], p['dec_emb']['temp_w'])

    d0 = p['dec0']
    L_DEC = LABEL_LEN + PRED_LEN
    x = _attn_block(d0['self'], dec, dec, 2, True, d0['ln1_g'], d0['ln1_b'])
    x = _attn_block(d0['cross'], x, enc_out, 3, False, d0['ln2_g'], d0['ln2_b'])
    x2 = _ffn(x.reshape(-1, D_MODEL), d0['w1'].T, _row(d0['b1']),
              d0['w2'].T, _row(d0['b2']), _row(d0['ln3_g']), _row(d0['ln3_b']))

    xdec = x2.reshape(B, L_DEC, D_MODEL)[:, -PRED_LEN:, :]
    out = _final_proj(xdec.reshape(B * PRED_LEN, D_MODEL),
                      _row(p['dec_norm_g']), _row(p['dec_norm_b']),
                      p['proj_w'].T, _row(p['proj_b']))
    return out.reshape(B, PRED_LEN, C_OUT)


# speedup vs baseline: 5.3740x; 1.0759x over previous
import functools
import math

import jax
import jax.numpy as jnp
import numpy as np
from jax.experimental import pallas as pl
from jax.experimental.pallas import tpu as pltpu

B = 4
SEQ = 2048
ENC_IN = 7
C_OUT = 7
MARK_DIM = 4
D_MODEL = 256
N_HEADS = 8
HD = D_MODEL // N_HEADS
D_FF = 1024
FACTOR = 5
LABEL_LEN = 1024
PRED_LEN = 1024

_NEG = -1e30


def _pos_embed_np(L, d):
    pos = np.arange(L)[:, None].astype(np.float64)
    div = np.exp(np.arange(0, d, 2) * -(np.log(10000.0) / d))
    pe = np.zeros((L, d))
    pe[:, 0::2] = np.sin(pos * div)
    pe[:, 1::2] = np.cos(pos * div)
    return pe.astype(np.float32)


_POS_2048 = _pos_embed_np(SEQ, D_MODEL)


def _u_part(Lk):
    return min(FACTOR * int(np.ceil(np.log(Lk))), Lk)


def _tf_rotl(x, d):
    return ((x << np.uint32(d)) | (x >> np.uint32(32 - d))).astype(np.uint32)


def _tf2x32(k0, k1, x0, x1):
    x0 = x0.astype(np.uint32).copy()
    x1 = x1.astype(np.uint32).copy()
    ks0, ks1 = np.uint32(k0), np.uint32(k1)
    ks2 = np.uint32(ks0 ^ ks1 ^ np.uint32(0x1BD11BDA))
    rot = [13, 15, 26, 6, 17, 29, 16, 24]
    x0 = (x0 + ks0).astype(np.uint32)
    x1 = (x1 + ks1).astype(np.uint32)
    keys = [(ks1, ks2), (ks2, ks0), (ks0, ks1), (ks1, ks2), (ks2, ks0)]
    for r in range(5):
        for d in (rot[:4] if r % 2 == 0 else rot[4:]):
            x0 = (x0 + x1).astype(np.uint32)
            x1 = _tf_rotl(x1, d)
            x1 = (x1 ^ x0).astype(np.uint32)
        ka, kb = keys[r]
        x0 = (x0 + ka).astype(np.uint32)
        x1 = (x1 + kb + np.uint32(r + 1)).astype(np.uint32)
    return x0, x1


def _tf_counter(n):
    cnt = np.arange(n, dtype=np.uint64)
    return ((cnt >> np.uint64(32)).astype(np.uint32),
            (cnt & np.uint64(0xFFFFFFFF)).astype(np.uint32))


def _tf_key(seed):
    return np.uint32(np.uint64(seed) >> np.uint64(32)), np.uint32(seed & 0xFFFFFFFF)


def _tf_fold_in(key, data):
    d0, d1 = _tf_key(int(data))
    x0, x1 = _tf2x32(key[0], key[1], np.array([d0]), np.array([d1]))
    return np.uint32(x0[0]), np.uint32(x1[0])


def _tf_randint_pow2(key, shape, span):
    c0, c1 = _tf_counter(2)
    s0, s1 = _tf2x32(key[0], key[1], c0, c1)
    k2 = (s0[1], s1[1])
    n = int(np.prod(shape))
    c0, c1 = _tf_counter(n)
    b0, b1 = _tf2x32(k2[0], k2[1], c0, c1)
    return ((b0 ^ b1) % np.uint32(span)).astype(np.int64).reshape(shape)


def _cnt_matrix_np(fold, LQ, LK):
    U = _u_part(LK)
    key = _tf_fold_in(_tf_key(42), fold)
    idx = _tf_randint_pow2(key, (LQ, U), LK)
    cnt = np.zeros((LQ, LK), np.int8)
    np.add.at(cnt, (np.arange(LQ)[:, None], idx), 1)
    return cnt


_CNTS_T = {
    0: np.ascontiguousarray(_cnt_matrix_np(0, SEQ, SEQ).T),
    1: np.ascontiguousarray(_cnt_matrix_np(1, SEQ // 2, SEQ // 2).T),
    2: np.ascontiguousarray(_cnt_matrix_np(2, SEQ, SEQ).T),
    3: np.ascontiguousarray(_cnt_matrix_np(3, SEQ, SEQ // 2).T),
}


def _embed_krn(xc_ref, w_ref, pos_ref, o_ref):
    o_ref[0] = jnp.dot(xc_ref[0], w_ref[...],
                       preferred_element_type=jnp.float32) + pos_ref[...]


def _embed(xc, w, pos):
    b, L, kin = xc.shape
    TL = 256
    return pl.pallas_call(
        _embed_krn,
        grid=(b, L // TL),
        in_specs=[
            pl.BlockSpec((1, TL, kin), lambda i, j: (i, j, 0)),
            pl.BlockSpec((kin, D_MODEL), lambda i, j: (0, 0)),
            pl.BlockSpec((TL, D_MODEL), lambda i, j: (j, 0)),
        ],
        out_specs=pl.BlockSpec((1, TL, D_MODEL), lambda i, j: (i, j, 0)),
        out_shape=jax.ShapeDtypeStruct((b, L, D_MODEL), jnp.float32),
    )(xc, w, pos)


def _qkv_krn(x_ref, w_ref, b_ref, *o_refs):
    y = jnp.dot(x_ref[0], w_ref[...],
                preferred_element_type=jnp.float32) + b_ref[...]
    for t, o_ref in enumerate(o_refs):
        for h in range(N_HEADS):
            c = t * D_MODEL + h * HD
            o_ref[0, h] = y[:, c:c + HD]


def _embed_qkv_krn(xc_ref, we_ref, pos_ref, w_ref, b_ref,
                   e_ref, *o_refs):
    emb = jnp.dot(xc_ref[0], we_ref[...],
                  preferred_element_type=jnp.float32) + pos_ref[...]
    e_ref[0] = emb
    y = jnp.dot(emb, w_ref[...],
                preferred_element_type=jnp.float32) + b_ref[...]
    for t, o_ref in enumerate(o_refs):
        for h in range(N_HEADS):
            c = t * D_MODEL + h * HD
            o_ref[0, h] = y[:, c:c + HD]


def _embed_qkv(xc, we, pos, w, b):
    b_, L, kin = xc.shape
    TL = 256
    emb, q, k, v = pl.pallas_call(
        _embed_qkv_krn,
        grid=(b_, L // TL),
        in_specs=[
            pl.BlockSpec((1, TL, kin), lambda i, j: (i, j, 0)),
            pl.BlockSpec((kin, D_MODEL), lambda i, j: (0, 0)),
            pl.BlockSpec((TL, D_MODEL), lambda i, j: (j, 0)),
            pl.BlockSpec((D_MODEL, 3 * D_MODEL), lambda i, j: (0, 0)),
            pl.BlockSpec((1, 3 * D_MODEL), lambda i, j: (0, 0)),
        ],
        out_specs=[pl.BlockSpec((1, TL, D_MODEL), lambda i, j: (i, j, 0))]
        + [pl.BlockSpec((1, N_HEADS, TL, HD), lambda i, j: (i, 0, j, 0))
           for _ in range(3)],
        out_shape=[jax.ShapeDtypeStruct((b_, L, D_MODEL), jnp.float32)]
        + [jax.ShapeDtypeStruct((b_, N_HEADS, L, HD), jnp.float32)
           for _ in range(3)],
    )(xc, we, pos, w, b)
    return emb, q, k, v


def _qkv_proj(x, w, b, n_out):
    b_, L, K = x.shape
    TL = 256
    outs = pl.pallas_call(
        _qkv_krn,
        grid=(b_, L // TL),
        in_specs=[
            pl.BlockSpec((1, TL, K), lambda i, j: (i, j, 0)),
            pl.BlockSpec((K, n_out * D_MODEL), lambda i, j: (0, 0)),
            pl.BlockSpec((1, n_out * D_MODEL), lambda i, j: (0, 0)),
        ],
        out_specs=[pl.BlockSpec((1, N_HEADS, TL, HD), lambda i, j: (i, 0, j, 0))
                   for _ in range(n_out)],
        out_shape=[jax.ShapeDtypeStruct((b_, N_HEADS, L, HD), jnp.float32)
                   for _ in range(n_out)],
    )(x, w, b)
    return outs


def _ln(y, g, b):
    m = jnp.mean(y, axis=-1, keepdims=True)
    v = jnp.mean((y - m) ** 2, axis=-1, keepdims=True)
    return (y - m) * jax.lax.rsqrt(v + 1e-5) * g + b


def _proj_res_ln_krn(x_ref, r_ref, w_ref, b_ref, g_ref, be_ref, o_ref):
    x = jnp.concatenate([x_ref[0, h] for h in range(N_HEADS)], axis=1)
    y = jnp.dot(x, w_ref[...],
                preferred_element_type=jnp.float32) + b_ref[...] + r_ref[...]
    o_ref[...] = _ln(y, g_ref[...], be_ref[...])


def _proj_res_ln(ctx, resid, w, b, g, beta, b_, L):
    N = w.shape[1]
    TL = 256
    nl = L // TL
    return pl.pallas_call(
        _proj_res_ln_krn,
        grid=(b_, nl),
        in_specs=[
            pl.BlockSpec((1, N_HEADS, TL, HD), lambda i, j: (i, 0, j, 0)),
            pl.BlockSpec((TL, N), lambda i, j: (i * nl + j, 0)),
            pl.BlockSpec((D_MODEL, N), lambda i, j: (0, 0)),
            pl.BlockSpec((1, N), lambda i, j: (0, 0)),
            pl.BlockSpec((1, N), lambda i, j: (0, 0)),
            pl.BlockSpec((1, N), lambda i, j: (0, 0)),
        ],
        out_specs=pl.BlockSpec((TL, N), lambda i, j: (i * nl + j, 0)),
        out_shape=jax.ShapeDtypeStruct((b_ * L, N), jnp.float32),
    )(ctx, resid, w, b, g, beta)


def _gelu_exact(x):
    return x * 0.5 * (1.0 + jax.lax.erf(x * (1.0 / math.sqrt(2.0))))


def _proj_ffn_krn(x_ref, r_ref, wo_ref, bo_ref, g1_ref, be1_ref,
                  w1_ref, b1_ref, w2_ref, b2_ref, g2_ref, be2_ref,
                  g3_ref, be3_ref, o_ref, *, three_ln):
    x = jnp.concatenate([x_ref[0, h] for h in range(N_HEADS)], axis=1)
    y = jnp.dot(x, wo_ref[...],
                preferred_element_type=jnp.float32) + bo_ref[...] + r_ref[...]
    y = _ln(y, g1_ref[...], be1_ref[...])
    h = _gelu_exact(jnp.dot(y, w1_ref[...],
                            preferred_element_type=jnp.float32) + b1_ref[...])
    z = y + jnp.dot(h, w2_ref[...],
                    preferred_element_type=jnp.float32) + b2_ref[...]
    z = _ln(z, g2_ref[...], be2_ref[...])
    if three_ln:
        z = _ln(z, g3_ref[...], be3_ref[...])
    o_ref[...] = z


def _proj_ffn(ctx, resid, wo, bo, g1, be1, w1, b1, w2, b2, g2, be2,
              b_, L, g3=None, be3=None):
    three_ln = g3 is not None
    if g3 is None:
        g3, be3 = g2, be2
    TL = 256
    nl = L // TL
    return pl.pallas_call(
        functools.partial(_proj_ffn_krn, three_ln=three_ln),
        grid=(b_, nl),
        in_specs=[
            pl.BlockSpec((1, N_HEADS, TL, HD), lambda i, j: (i, 0, j, 0)),
            pl.BlockSpec((TL, D_MODEL), lambda i, j: (i * nl + j, 0)),
            pl.BlockSpec((D_MODEL, D_MODEL), lambda i, j: (0, 0)),
            pl.BlockSpec((1, D_MODEL), lambda i, j: (0, 0)),
            pl.BlockSpec((1, D_MODEL), lambda i, j: (0, 0)),
            pl.BlockSpec((1, D_MODEL), lambda i, j: (0, 0)),
            pl.BlockSpec((D_MODEL, D_FF), lambda i, j: (0, 0)),
            pl.BlockSpec((1, D_FF), lambda i, j: (0, 0)),
            pl.BlockSpec((D_FF, D_MODEL), lambda i, j: (0, 0)),
            pl.BlockSpec((1, D_MODEL), lambda i, j: (0, 0)),
            pl.BlockSpec((1, D_MODEL), lambda i, j: (0, 0)),
            pl.BlockSpec((1, D_MODEL), lambda i, j: (0, 0)),
            pl.BlockSpec((1, D_MODEL), lambda i, j: (0, 0)),
            pl.BlockSpec((1, D_MODEL), lambda i, j: (0, 0)),
        ],
        out_specs=pl.BlockSpec((TL, D_MODEL), lambda i, j: (i * nl + j, 0)),
        out_shape=jax.ShapeDtypeStruct((b_ * L, D_MODEL), jnp.float32),
    )(ctx, resid, wo, bo, g1, be1, w1, b1, w2, b2, g2, be2, g3, be3)


def _ffn_krn(x_ref, w1_ref, b1_ref, w2_ref, b2_ref, g_ref, be_ref,
             g2_ref, be2_ref, o_ref, *, two_ln):
    x = x_ref[...]
    h = _gelu_exact(jnp.dot(x, w1_ref[...],
                            preferred_element_type=jnp.float32) + b1_ref[...])
    y = x + jnp.dot(h, w2_ref[...],
                    preferred_element_type=jnp.float32) + b2_ref[...]
    y = _ln(y, g_ref[...], be_ref[...])
    if two_ln:
        y = _ln(y, g2_ref[...], be2_ref[...])
    o_ref[...] = y


def _ffn(x, w1, b1, w2, b2, g, beta, g2=None, beta2=None):
    M, K = x.shape
    two_ln = g2 is not None
    if g2 is None:
        g2, beta2 = g, beta
    TM = 256
    return pl.pallas_call(
        functools.partial(_ffn_krn, two_ln=two_ln),
        grid=(M // TM,),
        in_specs=[
            pl.BlockSpec((TM, K), lambda i: (i, 0)),
            pl.BlockSpec((K, D_FF), lambda i: (0, 0)),
            pl.BlockSpec((1, D_FF), lambda i: (0, 0)),
            pl.BlockSpec((D_FF, K), lambda i: (0, 0)),
            pl.BlockSpec((1, K), lambda i: (0, 0)),
            pl.BlockSpec((1, K), lambda i: (0, 0)),
            pl.BlockSpec((1, K), lambda i: (0, 0)),
            pl.BlockSpec((1, K), lambda i: (0, 0)),
            pl.BlockSpec((1, K), lambda i: (0, 0)),
        ],
        out_specs=pl.BlockSpec((TM, K), lambda i: (i, 0)),
        out_shape=jax.ShapeDtypeStruct((M, K), jnp.float32),
    )(x, w1, b1, w2, b2, g, beta, g2, beta2)


def _fattn_krn(q_ref, k_ref, v_ref, ct_ref, o_ref, m_ref, p_ref, cm_ref,
               *, u, upad, LQ, LK, masked, HG):
    TQ = 256

    for t in range(LQ // TQ):
        cf = ct_ref[:, t * TQ:(t + 1) * TQ].astype(jnp.float32)
        madd = jnp.where(cf > 0.5, 0.0, _NEG)
        for h in range(HG):
            qt = q_ref[0, h, t * TQ:(t + 1) * TQ, :]
            st = jax.lax.dot_general(k_ref[0, h], qt, (((1,), (1,)), ((), ())),
                                     preferred_element_type=jnp.float32)
            mx = jnp.max(st + madd, axis=0, keepdims=True)
            sm = jnp.sum(st * cf, axis=0, keepdims=True) * (1.0 / LK)
            m_ref[h:h + 1, t * TQ:(t + 1) * TQ] = mx - sm

    m = m_ref[...]
    lane_q = jax.lax.broadcasted_iota(jnp.int32, (HG, LQ), 1)
    if masked:
        lane_k = jax.lax.broadcasted_iota(jnp.int32, (HG, LK), 1)
    for i in range(u):
        mx = jnp.max(m, axis=-1, keepdims=True)
        am = jnp.min(jnp.where(m == mx, lane_q, LQ), axis=-1, keepdims=True)
        p_ref[:, i, :] = (lane_q == am).astype(jnp.float32)
        if masked:
            cm_ref[:, i, :] = (lane_k > am).astype(jnp.float32)
        m = jnp.where(lane_q == am, _NEG, m)
    for i in range(u, upad):
        p_ref[:, i, :] = jnp.zeros((HG, LQ), jnp.float32)
        if masked:
            cm_ref[:, i, :] = jnp.zeros((HG, LK), jnp.float32)

    ones = jnp.ones((upad, HD), jnp.float32)
    if masked:
        TT = 256
        r_io = jax.lax.broadcasted_iota(jnp.int32, (TT, TT), 0)
        c_io = jax.lax.broadcasted_iota(jnp.int32, (TT, TT), 1)
        tri = (r_io >= c_io).astype(jnp.float32)
    for h in range(HG):
        V = v_ref[0, h]
        ph = p_ref[h]
        qred = jnp.dot(ph, q_ref[0, h], preferred_element_type=jnp.float32)
        s = jax.lax.dot_general(qred, k_ref[0, h], (((1,), (1,)), ((), ())),
                                preferred_element_type=jnp.float32)
        s = s * (1.0 / math.sqrt(HD))
        if masked:
            s = s + cm_ref[h] * _NEG
        mx = jnp.max(s, axis=-1, keepdims=True)
        e = jnp.exp(s - mx)
        p = e / jnp.sum(e, axis=-1, keepdims=True)
        upd = jnp.dot(p, V, preferred_element_type=jnp.float32)
        ctxsel = jax.lax.dot_general(ph, upd, (((0,), (0,)), ((), ())),
                                     preferred_element_type=jnp.float32)
        covm = jax.lax.dot_general(ph, ones, (((0,), (0,)), ((), ())),
                                   preferred_element_type=jnp.float32)
        if masked:
            carry = jnp.zeros((1, HD), jnp.float32)
            for t in range(LK // TT):
                vt = V[t * TT:(t + 1) * TT, :]
                cum = jnp.dot(tri, vt,
                              preferred_element_type=jnp.float32) + carry
                sl = slice(t * TT, (t + 1) * TT)
                o_ref[0, h, sl, :] = (cum * (1.0 - covm[sl, :])
                                      + ctxsel[sl, :])
                carry = cum[TT - 1:TT, :]
        else:
            mean = jnp.sum(V, axis=0, keepdims=True) * (1.0 / LK)
            base = jnp.broadcast_to(mean, (LQ, HD))
            o_ref[0, h] = base * (1.0 - covm) + ctxsel


def _fused_attn(q, k, v, cntT, u, masked):
    b, _, LQ, _ = q.shape
    LK = k.shape[2]
    upad = ((u + 7) // 8) * 8
    HG = 4
    scratch = [
        pltpu.VMEM((HG, LQ), jnp.float32),
        pltpu.VMEM((HG, upad, LQ), jnp.float32),
        pltpu.VMEM((HG, upad, LK), jnp.float32),
    ]
    return pl.pallas_call(
        functools.partial(_fattn_krn, u=u, upad=upad, LQ=LQ, LK=LK,
                          masked=masked, HG=HG),
        grid=(b, N_HEADS // HG),
        in_specs=[
            pl.BlockSpec((1, HG, LQ, HD), lambda g, j: (g, j, 0, 0)),
            pl.BlockSpec((1, HG, LK, HD), lambda g, j: (g, j, 0, 0)),
            pl.BlockSpec((1, HG, LK, HD), lambda g, j: (g, j, 0, 0)),
            pl.BlockSpec((LK, LQ), lambda g, j: (0, 0)),
        ],
        out_specs=pl.BlockSpec((1, HG, LQ, HD), lambda g, j: (g, j, 0, 0)),
        out_shape=jax.ShapeDtypeStruct((b, N_HEADS, LQ, HD), jnp.float32),
        scratch_shapes=scratch,
    )(q, k, v, cntT)


def _conv_distil_krn(x_ref, w0_ref, w1_ref, w2_ref, b_ref, sc_ref, bb_ref,
                     o_ref, *, L):
    x = x_ref[0]
    xm1 = jnp.concatenate([x[L - 1:L, :], x[:L - 1, :]], axis=0)
    xp1 = jnp.concatenate([x[1:, :], x[:1, :]], axis=0)
    y = (jnp.dot(xm1, w0_ref[...], preferred_element_type=jnp.float32)
         + jnp.dot(x, w1_ref[...], preferred_element_type=jnp.float32)
         + jnp.dot(xp1, w2_ref[...], preferred_element_type=jnp.float32)
         + b_ref[...])
    y = y * sc_ref[...] + bb_ref[...]
    y = jnp.where(y > 0, y, jnp.exp(jnp.minimum(y, 0.0)) - 1.0)
    C = y.shape[1]
    neg = jnp.full((1, C), _NEG, jnp.float32)
    ym1 = jnp.concatenate([neg, y[:L - 1, :]], axis=0)
    yp1 = jnp.concatenate([y[1:, :], neg], axis=0)
    pf = jnp.maximum(jnp.maximum(ym1, y), yp1)
    o_ref[0] = pf


def _conv_distil(x, w0, w1, w2, b, scale, bb):
    b_, L, C = x.shape
    full = pl.pallas_call(
        functools.partial(_conv_distil_krn, L=L),
        grid=(b_,),
        in_specs=[
            pl.BlockSpec((1, L, C), lambda i: (i, 0, 0)),
            pl.BlockSpec((C, C), lambda i: (0, 0)),
            pl.BlockSpec((C, C), lambda i: (0, 0)),
            pl.BlockSpec((C, C), lambda i: (0, 0)),
            pl.BlockSpec((1, C), lambda i: (0, 0)),
            pl.BlockSpec((1, C), lambda i: (0, 0)),
            pl.BlockSpec((1, C), lambda i: (0, 0)),
        ],
        out_specs=pl.BlockSpec((1, L, C), lambda i: (i, 0, 0)),
        out_shape=jax.ShapeDtypeStruct((b_, L, C), jnp.float32),
    )(x, w0, w1, w2, b, scale, bb)
    return full[:, ::2, :]


def _final_krn(x_ref, g_ref, be_ref, w_ref, b_ref, o_ref):
    y = _ln(x_ref[...], g_ref[...], be_ref[...])
    o_ref[...] = jnp.dot(y, w_ref[...],
                         preferred_element_type=jnp.float32) + b_ref[...]


def _final_proj(x, g, beta, w, b):
    M, K = x.shape
    N = w.shape[1]
    TM = 256
    return pl.pallas_call(
        _final_krn,
        grid=(M // TM,),
        in_specs=[
            pl.BlockSpec((TM, K), lambda i: (i, 0)),
            pl.BlockSpec((1, K), lambda i: (0, 0)),
            pl.BlockSpec((1, K), lambda i: (0, 0)),
            pl.BlockSpec((K, N), lambda i: (0, 0)),
            pl.BlockSpec((1, N), lambda i: (0, 0)),
        ],
        out_specs=pl.BlockSpec((TM, N), lambda i: (i, 0)),
        out_shape=jax.ShapeDtypeStruct((M, N), jnp.float32),
    )(x, g, beta, w, b)


def _row(v):
    return v.reshape(1, -1)


def _build_xc(x, mark, conv_w, temp_w):
    xc = jnp.concatenate(
        [jnp.roll(x, 1, axis=1), x, jnp.roll(x, -1, axis=1), mark], axis=-1)
    kin = xc.shape[-1]
    pad = (-kin) % 32
    xc = jnp.pad(xc, ((0, 0), (0, 0), (0, pad)))
    w = jnp.concatenate([conv_w[:, :, 0].T, conv_w[:, :, 1].T,
                         conv_w[:, :, 2].T, temp_w.T], axis=0)
    w = jnp.pad(w, ((0, pad), (0, 0)))
    return xc, w


def _qkv_cat(p, names):
    w = jnp.concatenate([p['w' + n].T for n in names], axis=1)
    b = jnp.concatenate([p['b' + n] for n in names])
    return w, _row(b)


def _u_of(LQ):
    return min(FACTOR * int(np.ceil(np.log(LQ))), LQ)


def kernel(batch_x, batch_x_mark, batch_y_mark, params):
    p = params
    pos = jnp.asarray(_POS_2048)

    e0 = p['enc0']
    xc, we = _build_xc(batch_x, batch_x_mark,
                       p['enc_emb']['conv_w'], p['enc_emb']['temp_w'])
    wqkv, bqkv = _qkv_cat(e0['attn'], ['q', 'k', 'v'])
    emb, q, k, v = _embed_qkv(xc, we, pos, wqkv, bqkv)

    ctx = _fused_attn(q, k, v, jnp.asarray(_CNTS_T[0]), _u_of(SEQ), False)
    x2 = _proj_ffn(ctx, emb.reshape(-1, D_MODEL), e0['attn']['wo'].T,
                   _row(e0['attn']['bo']), _row(e0['ln1_g']), _row(e0['ln1_b']),
                   e0['w1'].T, _row(e0['b1']), e0['w2'].T, _row(e0['b2']),
                   _row(e0['ln2_g']), _row(e0['ln2_b']), B, SEQ)

    c0 = p['conv0']
    scale = _row(c0['bn_g'] * (1.0 / np.sqrt(1.0 + 1e-5)))
    x = _conv_distil(x2.reshape(B, SEQ, D_MODEL), c0['w'][:, :, 0].T,
                     c0['w'][:, :, 1].T, c0['w'][:, :, 2].T, _row(c0['b']),
                     scale, _row(c0['bn_b']))

    e1 = p['enc1']
    L1 = SEQ // 2
    wqkv, bqkv = _qkv_cat(e1['attn'], ['q', 'k', 'v'])
    q, k, v = _qkv_proj(x, wqkv, bqkv, 3)
    ctx = _fused_attn(q, k, v, jnp.asarray(_CNTS_T[1]), _u_of(L1), False)
    x2 = _proj_ffn(ctx, x.reshape(-1, D_MODEL), e1['attn']['wo'].T,
                   _row(e1['attn']['bo']), _row(e1['ln1_g']), _row(e1['ln1_b']),
                   e1['w1'].T, _row(e1['b1']), e1['w2'].T, _row(e1['b2']),
                   _row(e1['ln2_g']), _row(e1['ln2_b']), B, L1,
                   _row(p['enc_norm_g']), _row(p['enc_norm_b']))
    enc_out = x2.reshape(B, L1, D_MODEL)

    d0 = p['dec0']
    L_DEC = LABEL_LEN + PRED_LEN
    dec_inp = jnp.concatenate(
        [batch_x[:, -LABEL_LEN:, :],
         jnp.zeros((B, PRED_LEN, ENC_IN), jnp.float32)], axis=1)
    xc, we = _build_xc(dec_inp, batch_y_mark,
                       p['dec_emb']['conv_w'], p['dec_emb']['temp_w'])
    wqkv, bqkv = _qkv_cat(d0['self'], ['q', 'k', 'v'])
    emb, q, k, v = _embed_qkv(xc, we, pos, wqkv, bqkv)

    ctx = _fused_attn(q, k, v, jnp.asarray(_CNTS_T[2]), _u_of(L_DEC), True)
    t = _proj_res_ln(ctx, emb.reshape(-1, D_MODEL), d0['self']['wo'].T,
                     _row(d0['self']['bo']), _row(d0['ln1_g']),
                     _row(d0['ln1_b']), B, L_DEC)

    q2, = _qkv_proj(t.reshape(B, L_DEC, D_MODEL), d0['cross']['wq'].T,
                    _row(d0['cross']['bq']), 1)
    wkv, bkv = _qkv_cat(d0['cross'], ['k', 'v'])
    k2, v2 = _qkv_proj(enc_out, wkv, bkv, 2)
    ctx2 = _fused_attn(q2, k2, v2, jnp.asarray(_CNTS_T[3]), _u_of(L_DEC), False)
    x2 = _proj_ffn(ctx2, t, d0['cross']['wo'].T, _row(d0['cross']['bo']),
                   _row(d0['ln2_g']), _row(d0['ln2_b']),
                   d0['w1'].T, _row(d0['b1']), d0['w2'].T, _row(d0['b2']),
                   _row(d0['ln3_g']), _row(d0['ln3_b']), B, L_DEC)

    xdec = x2.reshape(B, L_DEC, D_MODEL)[:, -PRED_LEN:, :]
    out = _final_proj(xdec.reshape(B * PRED_LEN, D_MODEL),
                      _row(p['dec_norm_g']), _row(p['dec_norm_b']),
                      p['proj_w'].T, _row(p['proj_b']))
    return out.reshape(B, PRED_LEN, C_OUT)


# speedup vs baseline: 5.4998x; 1.0234x over previous
import functools
import math

import jax
import jax.numpy as jnp
import numpy as np
from jax.experimental import pallas as pl
from jax.experimental.pallas import tpu as pltpu

B = 4
SEQ = 2048
ENC_IN = 7
C_OUT = 7
MARK_DIM = 4
D_MODEL = 256
N_HEADS = 8
HD = D_MODEL // N_HEADS
D_FF = 1024
FACTOR = 5
LABEL_LEN = 1024
PRED_LEN = 1024

_NEG = -1e30


def _pos_embed_np(L, d):
    pos = np.arange(L)[:, None].astype(np.float64)
    div = np.exp(np.arange(0, d, 2) * -(np.log(10000.0) / d))
    pe = np.zeros((L, d))
    pe[:, 0::2] = np.sin(pos * div)
    pe[:, 1::2] = np.cos(pos * div)
    return pe.astype(np.float32)


_POS_2048 = _pos_embed_np(SEQ, D_MODEL)


def _u_part(Lk):
    return min(FACTOR * int(np.ceil(np.log(Lk))), Lk)


def _tf_rotl(x, d):
    return ((x << np.uint32(d)) | (x >> np.uint32(32 - d))).astype(np.uint32)


def _tf2x32(k0, k1, x0, x1):
    x0 = x0.astype(np.uint32).copy()
    x1 = x1.astype(np.uint32).copy()
    ks0, ks1 = np.uint32(k0), np.uint32(k1)
    ks2 = np.uint32(ks0 ^ ks1 ^ np.uint32(0x1BD11BDA))
    rot = [13, 15, 26, 6, 17, 29, 16, 24]
    x0 = (x0 + ks0).astype(np.uint32)
    x1 = (x1 + ks1).astype(np.uint32)
    keys = [(ks1, ks2), (ks2, ks0), (ks0, ks1), (ks1, ks2), (ks2, ks0)]
    for r in range(5):
        for d in (rot[:4] if r % 2 == 0 else rot[4:]):
            x0 = (x0 + x1).astype(np.uint32)
            x1 = _tf_rotl(x1, d)
            x1 = (x1 ^ x0).astype(np.uint32)
        ka, kb = keys[r]
        x0 = (x0 + ka).astype(np.uint32)
        x1 = (x1 + kb + np.uint32(r + 1)).astype(np.uint32)
    return x0, x1


def _tf_counter(n):
    cnt = np.arange(n, dtype=np.uint64)
    return ((cnt >> np.uint64(32)).astype(np.uint32),
            (cnt & np.uint64(0xFFFFFFFF)).astype(np.uint32))


def _tf_key(seed):
    return np.uint32(np.uint64(seed) >> np.uint64(32)), np.uint32(seed & 0xFFFFFFFF)


def _tf_fold_in(key, data):
    d0, d1 = _tf_key(int(data))
    x0, x1 = _tf2x32(key[0], key[1], np.array([d0]), np.array([d1]))
    return np.uint32(x0[0]), np.uint32(x1[0])


def _tf_randint_pow2(key, shape, span):
    c0, c1 = _tf_counter(2)
    s0, s1 = _tf2x32(key[0], key[1], c0, c1)
    k2 = (s0[1], s1[1])
    n = int(np.prod(shape))
    c0, c1 = _tf_counter(n)
    b0, b1 = _tf2x32(k2[0], k2[1], c0, c1)
    return ((b0 ^ b1) % np.uint32(span)).astype(np.int64).reshape(shape)


def _cnt_matrix_np(fold, LQ, LK):
    U = _u_part(LK)
    key = _tf_fold_in(_tf_key(42), fold)
    idx = _tf_randint_pow2(key, (LQ, U), LK)
    cnt = np.zeros((LQ, LK), np.int8)
    np.add.at(cnt, (np.arange(LQ)[:, None], idx), 1)
    return cnt


_CNTS = {
    0: _cnt_matrix_np(0, SEQ, SEQ),
    1: _cnt_matrix_np(1, SEQ // 2, SEQ // 2),
    2: _cnt_matrix_np(2, SEQ, SEQ),
    3: _cnt_matrix_np(3, SEQ, SEQ // 2),
}


def _embed_krn(xc_ref, w_ref, pos_ref, o_ref):
    o_ref[0] = jnp.dot(xc_ref[0], w_ref[...],
                       preferred_element_type=jnp.float32) + pos_ref[...]


def _embed(xc, w, pos):
    b, L, kin = xc.shape
    TL = 256
    return pl.pallas_call(
        _embed_krn,
        grid=(b, L // TL),
        in_specs=[
            pl.BlockSpec((1, TL, kin), lambda i, j: (i, j, 0)),
            pl.BlockSpec((kin, D_MODEL), lambda i, j: (0, 0)),
            pl.BlockSpec((TL, D_MODEL), lambda i, j: (j, 0)),
        ],
        out_specs=pl.BlockSpec((1, TL, D_MODEL), lambda i, j: (i, j, 0)),
        out_shape=jax.ShapeDtypeStruct((b, L, D_MODEL), jnp.float32),
    )(xc, w, pos)


def _qkv_krn(x_ref, w_ref, b_ref, *o_refs, flat_off):
    y = jnp.dot(x_ref[0], w_ref[...],
                preferred_element_type=jnp.float32) + b_ref[...]
    n_split = len(o_refs) - (1 if flat_off is not None else 0)
    for t in range(n_split):
        for h in range(N_HEADS):
            c = t * D_MODEL + h * HD
            o_refs[t][0, h] = y[:, c:c + HD]
    if flat_off is not None:
        o_refs[-1][0] = y[:, flat_off:flat_off + D_MODEL]


def _embed_qkv_krn(xc_ref, we_ref, pos_ref, w_ref, b_ref,
                   e_ref, *o_refs):
    emb = jnp.dot(xc_ref[0], we_ref[...],
                  preferred_element_type=jnp.float32) + pos_ref[...]
    e_ref[0] = emb
    y = jnp.dot(emb, w_ref[...],
                preferred_element_type=jnp.float32) + b_ref[...]
    for t in range(3):
        for h in range(N_HEADS):
            c = t * D_MODEL + h * HD
            o_refs[t][0, h] = y[:, c:c + HD]
    o_refs[3][0] = y[:, D_MODEL:2 * D_MODEL]


def _embed_qkv(xc, we, pos, w, b):
    b_, L, kin = xc.shape
    TL = 256
    emb, q, k, v, kflat = pl.pallas_call(
        _embed_qkv_krn,
        grid=(b_, L // TL),
        in_specs=[
            pl.BlockSpec((1, TL, kin), lambda i, j: (i, j, 0)),
            pl.BlockSpec((kin, D_MODEL), lambda i, j: (0, 0)),
            pl.BlockSpec((TL, D_MODEL), lambda i, j: (j, 0)),
            pl.BlockSpec((D_MODEL, 3 * D_MODEL), lambda i, j: (0, 0)),
            pl.BlockSpec((1, 3 * D_MODEL), lambda i, j: (0, 0)),
        ],
        out_specs=[pl.BlockSpec((1, TL, D_MODEL), lambda i, j: (i, j, 0))]
        + [pl.BlockSpec((1, N_HEADS, TL, HD), lambda i, j: (i, 0, j, 0))
           for _ in range(3)]
        + [pl.BlockSpec((1, TL, D_MODEL), lambda i, j: (i, j, 0))],
        out_shape=[jax.ShapeDtypeStruct((b_, L, D_MODEL), jnp.float32)]
        + [jax.ShapeDtypeStruct((b_, N_HEADS, L, HD), jnp.float32)
           for _ in range(3)]
        + [jax.ShapeDtypeStruct((b_, L, D_MODEL), jnp.float32)],
    )(xc, we, pos, w, b)
    return emb, q, k, v, kflat


def _qkv_proj(x, w, b, n_out, flat_off=None):
    b_, L, K = x.shape
    TL = 256
    out_specs = [pl.BlockSpec((1, N_HEADS, TL, HD), lambda i, j: (i, 0, j, 0))
                 for _ in range(n_out)]
    out_shape = [jax.ShapeDtypeStruct((b_, N_HEADS, L, HD), jnp.float32)
                 for _ in range(n_out)]
    if flat_off is not None:
        out_specs.append(pl.BlockSpec((1, TL, D_MODEL), lambda i, j: (i, j, 0)))
        out_shape.append(jax.ShapeDtypeStruct((b_, L, D_MODEL), jnp.float32))
    outs = pl.pallas_call(
        functools.partial(_qkv_krn, flat_off=flat_off),
        grid=(b_, L // TL),
        in_specs=[
            pl.BlockSpec((1, TL, K), lambda i, j: (i, j, 0)),
            pl.BlockSpec((K, n_out * D_MODEL), lambda i, j: (0, 0)),
            pl.BlockSpec((1, n_out * D_MODEL), lambda i, j: (0, 0)),
        ],
        out_specs=out_specs,
        out_shape=out_shape,
    )(x, w, b)
    return outs


def _ln(y, g, b):
    m = jnp.mean(y, axis=-1, keepdims=True)
    v = jnp.mean((y - m) ** 2, axis=-1, keepdims=True)
    return (y - m) * jax.lax.rsqrt(v + 1e-5) * g + b


def _proj_res_ln_krn(x_ref, r_ref, w_ref, b_ref, g_ref, be_ref, o_ref):
    x = jnp.concatenate([x_ref[0, h] for h in range(N_HEADS)], axis=1)
    y = jnp.dot(x, w_ref[...],
                preferred_element_type=jnp.float32) + b_ref[...] + r_ref[...]
    o_ref[...] = _ln(y, g_ref[...], be_ref[...])


def _proj_res_ln(ctx, resid, w, b, g, beta, b_, L):
    N = w.shape[1]
    TL = 256
    nl = L // TL
    return pl.pallas_call(
        _proj_res_ln_krn,
        grid=(b_, nl),
        in_specs=[
            pl.BlockSpec((1, N_HEADS, TL, HD), lambda i, j: (i, 0, j, 0)),
            pl.BlockSpec((TL, N), lambda i, j: (i * nl + j, 0)),
            pl.BlockSpec((D_MODEL, N), lambda i, j: (0, 0)),
            pl.BlockSpec((1, N), lambda i, j: (0, 0)),
            pl.BlockSpec((1, N), lambda i, j: (0, 0)),
            pl.BlockSpec((1, N), lambda i, j: (0, 0)),
        ],
        out_specs=pl.BlockSpec((TL, N), lambda i, j: (i * nl + j, 0)),
        out_shape=jax.ShapeDtypeStruct((b_ * L, N), jnp.float32),
    )(ctx, resid, w, b, g, beta)


def _gelu_exact(x):
    return x * 0.5 * (1.0 + jax.lax.erf(x * (1.0 / math.sqrt(2.0))))


def _proj_ffn_krn(x_ref, r_ref, wo_ref, bo_ref, g1_ref, be1_ref,
                  w1_ref, b1_ref, w2_ref, b2_ref, g2_ref, be2_ref,
                  g3_ref, be3_ref, o_ref, *, three_ln):
    x = jnp.concatenate([x_ref[0, h] for h in range(N_HEADS)], axis=1)
    y = jnp.dot(x, wo_ref[...],
                preferred_element_type=jnp.float32) + bo_ref[...] + r_ref[...]
    y = _ln(y, g1_ref[...], be1_ref[...])
    h = _gelu_exact(jnp.dot(y, w1_ref[...],
                            preferred_element_type=jnp.float32) + b1_ref[...])
    z = y + jnp.dot(h, w2_ref[...],
                    preferred_element_type=jnp.float32) + b2_ref[...]
    z = _ln(z, g2_ref[...], be2_ref[...])
    if three_ln:
        z = _ln(z, g3_ref[...], be3_ref[...])
    o_ref[...] = z


def _proj_ffn(ctx, resid, wo, bo, g1, be1, w1, b1, w2, b2, g2, be2,
              b_, L, g3=None, be3=None):
    three_ln = g3 is not None
    if g3 is None:
        g3, be3 = g2, be2
    TL = 256
    nl = L // TL
    return pl.pallas_call(
        functools.partial(_proj_ffn_krn, three_ln=three_ln),
        grid=(b_, nl),
        in_specs=[
            pl.BlockSpec((1, N_HEADS, TL, HD), lambda i, j: (i, 0, j, 0)),
            pl.BlockSpec((TL, D_MODEL), lambda i, j: (i * nl + j, 0)),
            pl.BlockSpec((D_MODEL, D_MODEL), lambda i, j: (0, 0)),
            pl.BlockSpec((1, D_MODEL), lambda i, j: (0, 0)),
            pl.BlockSpec((1, D_MODEL), lambda i, j: (0, 0)),
            pl.BlockSpec((1, D_MODEL), lambda i, j: (0, 0)),
            pl.BlockSpec((D_MODEL, D_FF), lambda i, j: (0, 0)),
            pl.BlockSpec((1, D_FF), lambda i, j: (0, 0)),
            pl.BlockSpec((D_FF, D_MODEL), lambda i, j: (0, 0)),
            pl.BlockSpec((1, D_MODEL), lambda i, j: (0, 0)),
            pl.BlockSpec((1, D_MODEL), lambda i, j: (0, 0)),
            pl.BlockSpec((1, D_MODEL), lambda i, j: (0, 0)),
            pl.BlockSpec((1, D_MODEL), lambda i, j: (0, 0)),
            pl.BlockSpec((1, D_MODEL), lambda i, j: (0, 0)),
        ],
        out_specs=pl.BlockSpec((TL, D_MODEL), lambda i, j: (i * nl + j, 0)),
        out_shape=jax.ShapeDtypeStruct((b_ * L, D_MODEL), jnp.float32),
    )(ctx, resid, wo, bo, g1, be1, w1, b1, w2, b2, g2, be2, g3, be3)


def _ffn_krn(x_ref, w1_ref, b1_ref, w2_ref, b2_ref, g_ref, be_ref,
             g2_ref, be2_ref, o_ref, *, two_ln):
    x = x_ref[...]
    h = _gelu_exact(jnp.dot(x, w1_ref[...],
                            preferred_element_type=jnp.float32) + b1_ref[...])
    y = x + jnp.dot(h, w2_ref[...],
                    preferred_element_type=jnp.float32) + b2_ref[...]
    y = _ln(y, g_ref[...], be_ref[...])
    if two_ln:
        y = _ln(y, g2_ref[...], be2_ref[...])
    o_ref[...] = y


def _ffn(x, w1, b1, w2, b2, g, beta, g2=None, beta2=None):
    M, K = x.shape
    two_ln = g2 is not None
    if g2 is None:
        g2, beta2 = g, beta
    TM = 256
    return pl.pallas_call(
        functools.partial(_ffn_krn, two_ln=two_ln),
        grid=(M // TM,),
        in_specs=[
            pl.BlockSpec((TM, K), lambda i: (i, 0)),
            pl.BlockSpec((K, D_FF), lambda i: (0, 0)),
            pl.BlockSpec((1, D_FF), lambda i: (0, 0)),
            pl.BlockSpec((D_FF, K), lambda i: (0, 0)),
            pl.BlockSpec((1, K), lambda i: (0, 0)),
            pl.BlockSpec((1, K), lambda i: (0, 0)),
            pl.BlockSpec((1, K), lambda i: (0, 0)),
            pl.BlockSpec((1, K), lambda i: (0, 0)),
            pl.BlockSpec((1, K), lambda i: (0, 0)),
        ],
        out_specs=pl.BlockSpec((TM, K), lambda i: (i, 0)),
        out_shape=jax.ShapeDtypeStruct((M, K), jnp.float32),
    )(x, w1, b1, w2, b2, g, beta, g2, beta2)


def _fattn_krn(q_ref, k_ref, v_ref, kf_ref, c_ref, o_ref, mc_ref, p_ref,
               cm_ref, *, u, upad, LQ, LK, masked, HG):
    TQ = 256
    j = pl.program_id(1)

    for t in range(LQ // TQ):
        sl = slice(t * TQ, (t + 1) * TQ)
        cf = c_ref[sl, :].astype(jnp.float32)
        madd = jnp.where(cf > 0.5, 0.0, _NEG)
        sk = jnp.dot(cf, kf_ref[0], preferred_element_type=jnp.float32)
        for h in range(HG):
            qt = q_ref[0, h, sl, :]
            s = jax.lax.dot_general(qt, k_ref[0, h], (((1,), (1,)), ((), ())),
                                    preferred_element_type=jnp.float32)
            mxc = jnp.max(s + madd, axis=1, keepdims=True)
            smc = jnp.sum(qt * sk[:, h * HD:(h + 1) * HD], axis=1,
                          keepdims=True) * (1.0 / LK)
            mc_ref[sl, h:h + 1] = mxc - smc

    m = jnp.transpose(mc_ref[...])
    lane_q = jax.lax.broadcasted_iota(jnp.int32, (HG, LQ), 1)
    if masked:
        lane_k = jax.lax.broadcasted_iota(jnp.int32, (HG, LK), 1)
    for i in range(u):
        mx = jnp.max(m, axis=-1, keepdims=True)
        am = jnp.min(jnp.where(m == mx, lane_q, LQ), axis=-1, keepdims=True)
        p_ref[:, i, :] = (lane_q == am).astype(jnp.float32)
        if masked:
            cm_ref[:, i, :] = (lane_k > am).astype(jnp.float32)
        m = jnp.where(lane_q == am, _NEG, m)
    for i in range(u, upad):
        p_ref[:, i, :] = jnp.zeros((HG, LQ), jnp.float32)
        if masked:
            cm_ref[:, i, :] = jnp.zeros((HG, LK), jnp.float32)

    ones = jnp.ones((upad, HD), jnp.float32)
    if masked:
        TT = 256
        r_io = jax.lax.broadcasted_iota(jnp.int32, (TT, TT), 0)
        c_io = jax.lax.broadcasted_iota(jnp.int32, (TT, TT), 1)
        tri = (r_io >= c_io).astype(jnp.float32)
    for h in range(HG):
        V = v_ref[0, h]
        ph = p_ref[h]
        qred = jnp.dot(ph, q_ref[0, h], preferred_element_type=jnp.float32)
        s = jax.lax.dot_general(qred, k_ref[0, h], (((1,), (1,)), ((), ())),
                                preferred_element_type=jnp.float32)
        s = s * (1.0 / math.sqrt(HD))
        if masked:
            s = s + cm_ref[h] * _NEG
        mx = jnp.max(s, axis=-1, keepdims=True)
        e = jnp.exp(s - mx)
        p = e / jnp.sum(e, axis=-1, keepdims=True)
        upd = jnp.dot(p, V, preferred_element_type=jnp.float32)
        ctxsel = jax.lax.dot_general(ph, upd, (((0,), (0,)), ((), ())),
                                     preferred_element_type=jnp.float32)
        covm = jax.lax.dot_general(ph, ones, (((0,), (0,)), ((), ())),
                                   preferred_element_type=jnp.float32)
        if masked:
            carry = jnp.zeros((1, HD), jnp.float32)
            for t in range(LK // TT):
                vt = V[t * TT:(t + 1) * TT, :]
                cum = jnp.dot(tri, vt,
                              preferred_element_type=jnp.float32) + carry
                sl = slice(t * TT, (t + 1) * TT)
                o_ref[0, h, sl, :] = (cum * (1.0 - covm[sl, :])
                                      + ctxsel[sl, :])
                carry = cum[TT - 1:TT, :]
        else:
            mean = jnp.sum(V, axis=0, keepdims=True) * (1.0 / LK)
            base = jnp.broadcast_to(mean, (LQ, HD))
            o_ref[0, h] = base * (1.0 - covm) + ctxsel


def _fused_attn(q, k, v, kflat, cnt, u, masked):
    b, _, LQ, _ = q.shape
    LK = k.shape[2]
    upad = ((u + 7) // 8) * 8
    HG = 4
    scratch = [
        pltpu.VMEM((LQ, HG), jnp.float32),
        pltpu.VMEM((HG, upad, LQ), jnp.float32),
        pltpu.VMEM((HG, upad, LK), jnp.float32),
    ]
    return pl.pallas_call(
        functools.partial(_fattn_krn, u=u, upad=upad, LQ=LQ, LK=LK,
                          masked=masked, HG=HG),
        grid=(b, N_HEADS // HG),
        in_specs=[
            pl.BlockSpec((1, HG, LQ, HD), lambda g, j: (g, j, 0, 0)),
            pl.BlockSpec((1, HG, LK, HD), lambda g, j: (g, j, 0, 0)),
            pl.BlockSpec((1, HG, LK, HD), lambda g, j: (g, j, 0, 0)),
            pl.BlockSpec((1, LK, HG * HD), lambda g, j: (g, 0, j)),
            pl.BlockSpec((LQ, LK), lambda g, j: (0, 0)),
        ],
        out_specs=pl.BlockSpec((1, HG, LQ, HD), lambda g, j: (g, j, 0, 0)),
        out_shape=jax.ShapeDtypeStruct((b, N_HEADS, LQ, HD), jnp.float32),
        scratch_shapes=scratch,
    )(q, k, v, kflat, cnt)


def _conv_distil_krn(x_ref, w0_ref, w1_ref, w2_ref, b_ref, sc_ref, bb_ref,
                     o_ref, *, L):
    x = x_ref[0]
    xm1 = jnp.concatenate([x[L - 1:L, :], x[:L - 1, :]], axis=0)
    xp1 = jnp.concatenate([x[1:, :], x[:1, :]], axis=0)
    y = (jnp.dot(xm1, w0_ref[...], preferred_element_type=jnp.float32)
         + jnp.dot(x, w1_ref[...], preferred_element_type=jnp.float32)
         + jnp.dot(xp1, w2_ref[...], preferred_element_type=jnp.float32)
         + b_ref[...])
    y = y * sc_ref[...] + bb_ref[...]
    y = jnp.where(y > 0, y, jnp.exp(jnp.minimum(y, 0.0)) - 1.0)
    C = y.shape[1]
    neg = jnp.full((1, C), _NEG, jnp.float32)
    ym1 = jnp.concatenate([neg, y[:L - 1, :]], axis=0)
    yp1 = jnp.concatenate([y[1:, :], neg], axis=0)
    pf = jnp.maximum(jnp.maximum(ym1, y), yp1)
    r_io = jax.lax.broadcasted_iota(jnp.int32, (256, 512), 0)
    c_io = jax.lax.broadcasted_iota(jnp.int32, (256, 512), 1)
    dsel = (c_io == 2 * r_io).astype(jnp.float32)
    for t in range(L // 512):
        o_ref[0, t * 256:(t + 1) * 256, :] = jnp.dot(
            dsel, pf[t * 512:(t + 1) * 512, :],
            preferred_element_type=jnp.float32)


def _conv_distil(x, w0, w1, w2, b, scale, bb):
    b_, L, C = x.shape
    return pl.pallas_call(
        functools.partial(_conv_distil_krn, L=L),
        grid=(b_,),
        in_specs=[
            pl.BlockSpec((1, L, C), lambda i: (i, 0, 0)),
            pl.BlockSpec((C, C), lambda i: (0, 0)),
            pl.BlockSpec((C, C), lambda i: (0, 0)),
            pl.BlockSpec((C, C), lambda i: (0, 0)),
            pl.BlockSpec((1, C), lambda i: (0, 0)),
            pl.BlockSpec((1, C), lambda i: (0, 0)),
            pl.BlockSpec((1, C), lambda i: (0, 0)),
        ],
        out_specs=pl.BlockSpec((1, L // 2, C), lambda i: (i, 0, 0)),
        out_shape=jax.ShapeDtypeStruct((b_, L // 2, C), jnp.float32),
    )(x, w0, w1, w2, b, scale, bb)


def _final_krn(x_ref, g_ref, be_ref, w_ref, b_ref, o_ref):
    y = _ln(x_ref[...], g_ref[...], be_ref[...])
    o_ref[...] = jnp.dot(y, w_ref[...],
                         preferred_element_type=jnp.float32) + b_ref[...]


def _final_proj(x, g, beta, w, b):
    M, K = x.shape
    N = w.shape[1]
    TM = 256
    return pl.pallas_call(
        _final_krn,
        grid=(M // TM,),
        in_specs=[
            pl.BlockSpec((TM, K), lambda i: (i, 0)),
            pl.BlockSpec((1, K), lambda i: (0, 0)),
            pl.BlockSpec((1, K), lambda i: (0, 0)),
            pl.BlockSpec((K, N), lambda i: (0, 0)),
            pl.BlockSpec((1, N), lambda i: (0, 0)),
        ],
        out_specs=pl.BlockSpec((TM, N), lambda i: (i, 0)),
        out_shape=jax.ShapeDtypeStruct((M, N), jnp.float32),
    )(x, g, beta, w, b)


def _row(v):
    return v.reshape(1, -1)


def _build_xc(x, mark, conv_w, temp_w):
    xc = jnp.concatenate(
        [jnp.roll(x, 1, axis=1), x, jnp.roll(x, -1, axis=1), mark], axis=-1)
    kin = xc.shape[-1]
    pad = (-kin) % 32
    xc = jnp.pad(xc, ((0, 0), (0, 0), (0, pad)))
    w = jnp.concatenate([conv_w[:, :, 0].T, conv_w[:, :, 1].T,
                         conv_w[:, :, 2].T, temp_w.T], axis=0)
    w = jnp.pad(w, ((0, pad), (0, 0)))
    return xc, w


def _qkv_cat(p, names):
    w = jnp.concatenate([p['w' + n].T for n in names], axis=1)
    b = jnp.concatenate([p['b' + n] for n in names])
    return w, _row(b)


def _u_of(LQ):
    return min(FACTOR * int(np.ceil(np.log(LQ))), LQ)


def kernel(batch_x, batch_x_mark, batch_y_mark, params):
    p = params
    pos = jnp.asarray(_POS_2048)

    e0 = p['enc0']
    xc, we = _build_xc(batch_x, batch_x_mark,
                       p['enc_emb']['conv_w'], p['enc_emb']['temp_w'])
    wqkv, bqkv = _qkv_cat(e0['attn'], ['q', 'k', 'v'])
    emb, q, k, v, kflat = _embed_qkv(xc, we, pos, wqkv, bqkv)

    ctx = _fused_attn(q, k, v, kflat, jnp.asarray(_CNTS[0]), _u_of(SEQ), False)
    x2 = _proj_ffn(ctx, emb.reshape(-1, D_MODEL), e0['attn']['wo'].T,
                   _row(e0['attn']['bo']), _row(e0['ln1_g']), _row(e0['ln1_b']),
                   e0['w1'].T, _row(e0['b1']), e0['w2'].T, _row(e0['b2']),
                   _row(e0['ln2_g']), _row(e0['ln2_b']), B, SEQ)

    c0 = p['conv0']
    scale = _row(c0['bn_g'] * (1.0 / np.sqrt(1.0 + 1e-5)))
    x = _conv_distil(x2.reshape(B, SEQ, D_MODEL), c0['w'][:, :, 0].T,
                     c0['w'][:, :, 1].T, c0['w'][:, :, 2].T, _row(c0['b']),
                     scale, _row(c0['bn_b']))

    e1 = p['enc1']
    L1 = SEQ // 2
    wqkv, bqkv = _qkv_cat(e1['attn'], ['q', 'k', 'v'])
    q, k, v, kflat = _qkv_proj(x, wqkv, bqkv, 3, flat_off=D_MODEL)
    ctx = _fused_attn(q, k, v, kflat, jnp.asarray(_CNTS[1]), _u_of(L1), False)
    x2 = _proj_ffn(ctx, x.reshape(-1, D_MODEL), e1['attn']['wo'].T,
                   _row(e1['attn']['bo']), _row(e1['ln1_g']), _row(e1['ln1_b']),
                   e1['w1'].T, _row(e1['b1']), e1['w2'].T, _row(e1['b2']),
                   _row(e1['ln2_g']), _row(e1['ln2_b']), B, L1,
                   _row(p['enc_norm_g']), _row(p['enc_norm_b']))
    enc_out = x2.reshape(B, L1, D_MODEL)

    d0 = p['dec0']
    L_DEC = LABEL_LEN + PRED_LEN
    dec_inp = jnp.concatenate(
        [batch_x[:, -LABEL_LEN:, :],
         jnp.zeros((B, PRED_LEN, ENC_IN), jnp.float32)], axis=1)
    xc, we = _build_xc(dec_inp, batch_y_mark,
                       p['dec_emb']['conv_w'], p['dec_emb']['temp_w'])
    wqkv, bqkv = _qkv_cat(d0['self'], ['q', 'k', 'v'])
    emb, q, k, v, kflat = _embed_qkv(xc, we, pos, wqkv, bqkv)

    ctx = _fused_attn(q, k, v, kflat, jnp.asarray(_CNTS[2]), _u_of(L_DEC), True)
    t = _proj_res_ln(ctx, emb.reshape(-1, D_MODEL), d0['self']['wo'].T,
                     _row(d0['self']['bo']), _row(d0['ln1_g']),
                     _row(d0['ln1_b']), B, L_DEC)

    q2, = _qkv_proj(t.reshape(B, L_DEC, D_MODEL), d0['cross']['wq'].T,
                    _row(d0['cross']['bq']), 1)
    wkv, bkv = _qkv_cat(d0['cross'], ['k', 'v'])
    k2, v2, k2flat = _qkv_proj(enc_out, wkv, bkv, 2, flat_off=0)
    ctx2 = _fused_attn(q2, k2, v2, k2flat, jnp.asarray(_CNTS[3]), _u_of(L_DEC),
                       False)
    x2 = _proj_ffn(ctx2, t, d0['cross']['wo'].T, _row(d0['cross']['bo']),
                   _row(d0['ln2_g']), _row(d0['ln2_b']),
                   d0['w1'].T, _row(d0['b1']), d0['w2'].T, _row(d0['b2']),
                   _row(d0['ln3_g']), _row(d0['ln3_b']), B, L_DEC)

    xdec = x2.reshape(B, L_DEC, D_MODEL)[:, -PRED_LEN:, :]
    out = _final_proj(xdec.reshape(B * PRED_LEN, D_MODEL),
                      _row(p['dec_norm_g']), _row(p['dec_norm_b']),
                      p['proj_w'].T, _row(p['proj_b']))
    return out.reshape(B, PRED_LEN, C_OUT)


# speedup vs baseline: 5.8746x; 1.0681x over previous
import functools
import math

import jax
import jax.numpy as jnp
import numpy as np
from jax.experimental import pallas as pl
from jax.experimental.pallas import tpu as pltpu

B = 4
SEQ = 2048
ENC_IN = 7
C_OUT = 7
MARK_DIM = 4
D_MODEL = 256
N_HEADS = 8
HD = D_MODEL // N_HEADS
D_FF = 1024
FACTOR = 5
LABEL_LEN = 1024
PRED_LEN = 1024

_NEG = -1e30


def _pos_embed_np(L, d):
    pos = np.arange(L)[:, None].astype(np.float64)
    div = np.exp(np.arange(0, d, 2) * -(np.log(10000.0) / d))
    pe = np.zeros((L, d))
    pe[:, 0::2] = np.sin(pos * div)
    pe[:, 1::2] = np.cos(pos * div)
    return pe.astype(np.float32)


_POS_2048 = _pos_embed_np(SEQ, D_MODEL)


def _u_part(Lk):
    return min(FACTOR * int(np.ceil(np.log(Lk))), Lk)


def _tf_rotl(x, d):
    return ((x << np.uint32(d)) | (x >> np.uint32(32 - d))).astype(np.uint32)


def _tf2x32(k0, k1, x0, x1):
    x0 = x0.astype(np.uint32).copy()
    x1 = x1.astype(np.uint32).copy()
    ks0, ks1 = np.uint32(k0), np.uint32(k1)
    ks2 = np.uint32(ks0 ^ ks1 ^ np.uint32(0x1BD11BDA))
    rot = [13, 15, 26, 6, 17, 29, 16, 24]
    x0 = (x0 + ks0).astype(np.uint32)
    x1 = (x1 + ks1).astype(np.uint32)
    keys = [(ks1, ks2), (ks2, ks0), (ks0, ks1), (ks1, ks2), (ks2, ks0)]
    for r in range(5):
        for d in (rot[:4] if r % 2 == 0 else rot[4:]):
            x0 = (x0 + x1).astype(np.uint32)
            x1 = _tf_rotl(x1, d)
            x1 = (x1 ^ x0).astype(np.uint32)
        ka, kb = keys[r]
        x0 = (x0 + ka).astype(np.uint32)
        x1 = (x1 + kb + np.uint32(r + 1)).astype(np.uint32)
    return x0, x1


def _tf_counter(n):
    cnt = np.arange(n, dtype=np.uint64)
    return ((cnt >> np.uint64(32)).astype(np.uint32),
            (cnt & np.uint64(0xFFFFFFFF)).astype(np.uint32))


def _tf_key(seed):
    return np.uint32(np.uint64(seed) >> np.uint64(32)), np.uint32(seed & 0xFFFFFFFF)


def _tf_fold_in(key, data):
    d0, d1 = _tf_key(int(data))
    x0, x1 = _tf2x32(key[0], key[1], np.array([d0]), np.array([d1]))
    return np.uint32(x0[0]), np.uint32(x1[0])


def _tf_randint_pow2(key, shape, span):
    c0, c1 = _tf_counter(2)
    s0, s1 = _tf2x32(key[0], key[1], c0, c1)
    k2 = (s0[1], s1[1])
    n = int(np.prod(shape))
    c0, c1 = _tf_counter(n)
    b0, b1 = _tf2x32(k2[0], k2[1], c0, c1)
    return ((b0 ^ b1) % np.uint32(span)).astype(np.int64).reshape(shape)


def _cnt_matrix_np(fold, LQ, LK):
    U = _u_part(LK)
    key = _tf_fold_in(_tf_key(42), fold)
    idx = _tf_randint_pow2(key, (LQ, U), LK)
    cnt = np.zeros((LQ, LK), np.int8)
    np.add.at(cnt, (np.arange(LQ)[:, None], idx), 1)
    return cnt


_CNTS = {
    0: _cnt_matrix_np(0, SEQ, SEQ),
    1: _cnt_matrix_np(1, SEQ // 2, SEQ // 2),
    2: _cnt_matrix_np(2, SEQ, SEQ),
    3: _cnt_matrix_np(3, SEQ, SEQ // 2),
}


def _embed_krn(xc_ref, w_ref, pos_ref, o_ref):
    o_ref[0] = jnp.dot(xc_ref[0], w_ref[...],
                       preferred_element_type=jnp.float32) + pos_ref[...]


def _embed(xc, w, pos):
    b, L, kin = xc.shape
    TL = 256
    return pl.pallas_call(
        _embed_krn,
        grid=(b, L // TL),
        in_specs=[
            pl.BlockSpec((1, TL, kin), lambda i, j: (i, j, 0)),
            pl.BlockSpec((kin, D_MODEL), lambda i, j: (0, 0)),
            pl.BlockSpec((TL, D_MODEL), lambda i, j: (j, 0)),
        ],
        out_specs=pl.BlockSpec((1, TL, D_MODEL), lambda i, j: (i, j, 0)),
        out_shape=jax.ShapeDtypeStruct((b, L, D_MODEL), jnp.float32),
    )(xc, w, pos)


def _qkv_krn(x_ref, w_ref, b_ref, *o_refs):
    y = jnp.dot(x_ref[0], w_ref[...],
                preferred_element_type=jnp.float32) + b_ref[...]
    for t, o_ref in enumerate(o_refs):
        o_ref[0] = y[:, t * D_MODEL:(t + 1) * D_MODEL]


def _qkv_proj(x, w, b, n_out):
    b_, L, K = x.shape
    TL = 256
    outs = pl.pallas_call(
        _qkv_krn,
        grid=(b_, L // TL),
        in_specs=[
            pl.BlockSpec((1, TL, K), lambda i, j: (i, j, 0)),
            pl.BlockSpec((K, n_out * D_MODEL), lambda i, j: (0, 0)),
            pl.BlockSpec((1, n_out * D_MODEL), lambda i, j: (0, 0)),
        ],
        out_specs=[pl.BlockSpec((1, TL, D_MODEL), lambda i, j: (i, j, 0))
                   for _ in range(n_out)],
        out_shape=[jax.ShapeDtypeStruct((b_, L, D_MODEL), jnp.float32)
                   for _ in range(n_out)],
    )(x, w, b)
    return outs


def _embed_qkv_krn(xc_ref, we_ref, pos_ref, w_ref, b_ref, e_ref, *o_refs):
    emb = jnp.dot(xc_ref[0], we_ref[...],
                  preferred_element_type=jnp.float32) + pos_ref[...]
    e_ref[0] = emb
    y = jnp.dot(emb, w_ref[...],
                preferred_element_type=jnp.float32) + b_ref[...]
    for t, o_ref in enumerate(o_refs):
        o_ref[0] = y[:, t * D_MODEL:(t + 1) * D_MODEL]


def _embed_qkv(xc, we, pos, w, b):
    b_, L, kin = xc.shape
    TL = 256
    emb, q, k, v = pl.pallas_call(
        _embed_qkv_krn,
        grid=(b_, L // TL),
        in_specs=[
            pl.BlockSpec((1, TL, kin), lambda i, j: (i, j, 0)),
            pl.BlockSpec((kin, D_MODEL), lambda i, j: (0, 0)),
            pl.BlockSpec((TL, D_MODEL), lambda i, j: (j, 0)),
            pl.BlockSpec((D_MODEL, 3 * D_MODEL), lambda i, j: (0, 0)),
            pl.BlockSpec((1, 3 * D_MODEL), lambda i, j: (0, 0)),
        ],
        out_specs=[pl.BlockSpec((1, TL, D_MODEL), lambda i, j: (i, j, 0))
                   for _ in range(4)],
        out_shape=[jax.ShapeDtypeStruct((b_, L, D_MODEL), jnp.float32)
                   for _ in range(4)],
    )(xc, we, pos, w, b)
    return emb, q, k, v


def _ln(y, g, b):
    m = jnp.mean(y, axis=-1, keepdims=True)
    v = jnp.mean((y - m) ** 2, axis=-1, keepdims=True)
    return (y - m) * jax.lax.rsqrt(v + 1e-5) * g + b


def _proj_res_ln_krn(x_ref, r_ref, w_ref, b_ref, g_ref, be_ref, o_ref):
    y = jnp.dot(x_ref[...], w_ref[...],
                preferred_element_type=jnp.float32) + b_ref[...] + r_ref[...]
    o_ref[...] = _ln(y, g_ref[...], be_ref[...])


def _proj_res_ln(ctx, resid, w, b, g, beta, b_, L):
    N = w.shape[1]
    TL = 256
    nl = L // TL
    return pl.pallas_call(
        _proj_res_ln_krn,
        grid=(b_, nl),
        in_specs=[
            pl.BlockSpec((TL, N), lambda i, j: (i * nl + j, 0)),
            pl.BlockSpec((TL, N), lambda i, j: (i * nl + j, 0)),
            pl.BlockSpec((D_MODEL, N), lambda i, j: (0, 0)),
            pl.BlockSpec((1, N), lambda i, j: (0, 0)),
            pl.BlockSpec((1, N), lambda i, j: (0, 0)),
            pl.BlockSpec((1, N), lambda i, j: (0, 0)),
        ],
        out_specs=pl.BlockSpec((TL, N), lambda i, j: (i * nl + j, 0)),
        out_shape=jax.ShapeDtypeStruct((b_ * L, N), jnp.float32),
    )(ctx, resid, w, b, g, beta)


def _gelu_exact(x):
    return x * 0.5 * (1.0 + jax.lax.erf(x * (1.0 / math.sqrt(2.0))))


def _proj_ffn_krn(x_ref, r_ref, wo_ref, bo_ref, g1_ref, be1_ref,
                  w1_ref, b1_ref, w2_ref, b2_ref, g2_ref, be2_ref,
                  g3_ref, be3_ref, o_ref, *, three_ln):
    y = jnp.dot(x_ref[...], wo_ref[...],
                preferred_element_type=jnp.float32) + bo_ref[...] + r_ref[...]
    y = _ln(y, g1_ref[...], be1_ref[...])
    h = _gelu_exact(jnp.dot(y, w1_ref[...],
                            preferred_element_type=jnp.float32) + b1_ref[...])
    z = y + jnp.dot(h, w2_ref[...],
                    preferred_element_type=jnp.float32) + b2_ref[...]
    z = _ln(z, g2_ref[...], be2_ref[...])
    if three_ln:
        z = _ln(z, g3_ref[...], be3_ref[...])
    o_ref[...] = z


def _proj_ffn(ctx, resid, wo, bo, g1, be1, w1, b1, w2, b2, g2, be2,
              b_, L, g3=None, be3=None):
    three_ln = g3 is not None
    if g3 is None:
        g3, be3 = g2, be2
    TL = 256
    nl = L // TL
    return pl.pallas_call(
        functools.partial(_proj_ffn_krn, three_ln=three_ln),
        grid=(b_, nl),
        in_specs=[
            pl.BlockSpec((TL, D_MODEL), lambda i, j: (i * nl + j, 0)),
            pl.BlockSpec((TL, D_MODEL), lambda i, j: (i * nl + j, 0)),
            pl.BlockSpec((D_MODEL, D_MODEL), lambda i, j: (0, 0)),
            pl.BlockSpec((1, D_MODEL), lambda i, j: (0, 0)),
            pl.BlockSpec((1, D_MODEL), lambda i, j: (0, 0)),
            pl.BlockSpec((1, D_MODEL), lambda i, j: (0, 0)),
            pl.BlockSpec((D_MODEL, D_FF), lambda i, j: (0, 0)),
            pl.BlockSpec((1, D_FF), lambda i, j: (0, 0)),
            pl.BlockSpec((D_FF, D_MODEL), lambda i, j: (0, 0)),
            pl.BlockSpec((1, D_MODEL), lambda i, j: (0, 0)),
            pl.BlockSpec((1, D_MODEL), lambda i, j: (0, 0)),
            pl.BlockSpec((1, D_MODEL), lambda i, j: (0, 0)),
            pl.BlockSpec((1, D_MODEL), lambda i, j: (0, 0)),
            pl.BlockSpec((1, D_MODEL), lambda i, j: (0, 0)),
        ],
        out_specs=pl.BlockSpec((TL, D_MODEL), lambda i, j: (i * nl + j, 0)),
        out_shape=jax.ShapeDtypeStruct((b_ * L, D_MODEL), jnp.float32),
    )(ctx, resid, wo, bo, g1, be1, w1, b1, w2, b2, g2, be2, g3, be3)


def _ffn_krn(x_ref, w1_ref, b1_ref, w2_ref, b2_ref, g_ref, be_ref,
             g2_ref, be2_ref, o_ref, *, two_ln):
    x = x_ref[...]
    h = _gelu_exact(jnp.dot(x, w1_ref[...],
                            preferred_element_type=jnp.float32) + b1_ref[...])
    y = x + jnp.dot(h, w2_ref[...],
                    preferred_element_type=jnp.float32) + b2_ref[...]
    y = _ln(y, g_ref[...], be_ref[...])
    if two_ln:
        y = _ln(y, g2_ref[...], be2_ref[...])
    o_ref[...] = y


def _ffn(x, w1, b1, w2, b2, g, beta, g2=None, beta2=None):
    M, K = x.shape
    two_ln = g2 is not None
    if g2 is None:
        g2, beta2 = g, beta
    TM = 256
    return pl.pallas_call(
        functools.partial(_ffn_krn, two_ln=two_ln),
        grid=(M // TM,),
        in_specs=[
            pl.BlockSpec((TM, K), lambda i: (i, 0)),
            pl.BlockSpec((K, D_FF), lambda i: (0, 0)),
            pl.BlockSpec((1, D_FF), lambda i: (0, 0)),
            pl.BlockSpec((D_FF, K), lambda i: (0, 0)),
            pl.BlockSpec((1, K), lambda i: (0, 0)),
            pl.BlockSpec((1, K), lambda i: (0, 0)),
            pl.BlockSpec((1, K), lambda i: (0, 0)),
            pl.BlockSpec((1, K), lambda i: (0, 0)),
            pl.BlockSpec((1, K), lambda i: (0, 0)),
        ],
        out_specs=pl.BlockSpec((TM, K), lambda i: (i, 0)),
        out_shape=jax.ShapeDtypeStruct((M, K), jnp.float32),
    )(x, w1, b1, w2, b2, g, beta, g2, beta2)


def _fattn_krn(q_ref, k_ref, v_ref, c_ref, o_ref, mc_ref, p_ref,
               cm_ref, *, u, upad, LQ, LK, masked, HG):
    TQ = 256

    kb = k_ref[0]
    for t in range(LQ // TQ):
        sl = slice(t * TQ, (t + 1) * TQ)
        cf = c_ref[sl, :].astype(jnp.float32)
        madd = jnp.where(cf > 0.5, 0.0, _NEG)
        sk = jnp.dot(cf, kb, preferred_element_type=jnp.float32)
        for h in range(HG):
            hs = slice(h * HD, (h + 1) * HD)
            qt = q_ref[0, sl, hs]
            s = jax.lax.dot_general(qt, kb[:, hs], (((1,), (1,)), ((), ())),
                                    preferred_element_type=jnp.float32)
            mxc = jnp.max(s + madd, axis=1, keepdims=True)
            smc = jnp.sum(qt * sk[:, hs], axis=1,
                          keepdims=True) * (1.0 / LK)
            mc_ref[sl, h:h + 1] = mxc - smc

    m = jnp.transpose(mc_ref[...])
    lane_q = jax.lax.broadcasted_iota(jnp.int32, (HG, LQ), 1)
    if masked:
        lane_k = jax.lax.broadcasted_iota(jnp.int32, (HG, LK), 1)
    for i in range(u):
        mx = jnp.max(m, axis=-1, keepdims=True)
        am = jnp.min(jnp.where(m == mx, lane_q, LQ), axis=-1, keepdims=True)
        p_ref[:, i, :] = (lane_q == am).astype(jnp.float32)
        if masked:
            cm_ref[:, i, :] = (lane_k > am).astype(jnp.float32)
        m = jnp.where(lane_q == am, _NEG, m)
    for i in range(u, upad):
        p_ref[:, i, :] = jnp.zeros((HG, LQ), jnp.float32)
        if masked:
            cm_ref[:, i, :] = jnp.zeros((HG, LK), jnp.float32)

    ones = jnp.ones((upad, HD), jnp.float32)
    if masked:
        TT = 256
        r_io = jax.lax.broadcasted_iota(jnp.int32, (TT, TT), 0)
        c_io = jax.lax.broadcasted_iota(jnp.int32, (TT, TT), 1)
        tri = (r_io >= c_io).astype(jnp.float32)
    for h in range(HG):
        hs = slice(h * HD, (h + 1) * HD)
        V = v_ref[0, :, hs]
        ph = p_ref[h]
        qred = jnp.dot(ph, q_ref[0, :, hs],
                       preferred_element_type=jnp.float32)
        s = jax.lax.dot_general(qred, kb[:, hs], (((1,), (1,)), ((), ())),
                                preferred_element_type=jnp.float32)
        s = s * (1.0 / math.sqrt(HD))
        if masked:
            s = s + cm_ref[h] * _NEG
        mx = jnp.max(s, axis=-1, keepdims=True)
        e = jnp.exp(s - mx)
        p = e / jnp.sum(e, axis=-1, keepdims=True)
        upd = jnp.dot(p, V, preferred_element_type=jnp.float32)
        ctxsel = jax.lax.dot_general(ph, upd, (((0,), (0,)), ((), ())),
                                     preferred_element_type=jnp.float32)
        covm = jax.lax.dot_general(ph, ones, (((0,), (0,)), ((), ())),
                                   preferred_element_type=jnp.float32)
        if masked:
            carry = jnp.zeros((1, HD), jnp.float32)
            for t in range(LK // TT):
                vt = V[t * TT:(t + 1) * TT, :]
                cum = jnp.dot(tri, vt,
                              preferred_element_type=jnp.float32) + carry
                sl = slice(t * TT, (t + 1) * TT)
                o_ref[0, sl, hs] = (cum * (1.0 - covm[sl, :])
                                    + ctxsel[sl, :])
                carry = cum[TT - 1:TT, :]
        else:
            mean = jnp.sum(V, axis=0, keepdims=True) * (1.0 / LK)
            base = jnp.broadcast_to(mean, (LQ, HD))
            o_ref[0, :, hs] = base * (1.0 - covm) + ctxsel


def _fused_attn(q, k, v, cnt, u, masked):
    b, LQ, _ = q.shape
    LK = k.shape[1]
    upad = ((u + 7) // 8) * 8
    HG = 4
    W = HG * HD
    scratch = [
        pltpu.VMEM((LQ, HG), jnp.float32),
        pltpu.VMEM((HG, upad, LQ), jnp.float32),
        pltpu.VMEM((HG, upad, LK), jnp.float32),
    ]
    return pl.pallas_call(
        functools.partial(_fattn_krn, u=u, upad=upad, LQ=LQ, LK=LK,
                          masked=masked, HG=HG),
        grid=(b, N_HEADS // HG),
        in_specs=[
            pl.BlockSpec((1, LQ, W), lambda g, j: (g, 0, j)),
            pl.BlockSpec((1, LK, W), lambda g, j: (g, 0, j)),
            pl.BlockSpec((1, LK, W), lambda g, j: (g, 0, j)),
            pl.BlockSpec((LQ, LK), lambda g, j: (0, 0)),
        ],
        out_specs=pl.BlockSpec((1, LQ, W), lambda g, j: (g, 0, j)),
        out_shape=jax.ShapeDtypeStruct((b, LQ, D_MODEL), jnp.float32),
        scratch_shapes=scratch,
    )(q, k, v, cnt)


def _conv_distil_krn(x_ref, w0_ref, w1_ref, w2_ref, b_ref, sc_ref, bb_ref,
                     o_ref, *, L):
    x = x_ref[0]
    xm1 = jnp.concatenate([x[L - 1:L, :], x[:L - 1, :]], axis=0)
    xp1 = jnp.concatenate([x[1:, :], x[:1, :]], axis=0)
    y = (jnp.dot(xm1, w0_ref[...], preferred_element_type=jnp.float32)
         + jnp.dot(x, w1_ref[...], preferred_element_type=jnp.float32)
         + jnp.dot(xp1, w2_ref[...], preferred_element_type=jnp.float32)
         + b_ref[...])
    y = y * sc_ref[...] + bb_ref[...]
    y = jnp.where(y > 0, y, jnp.exp(jnp.minimum(y, 0.0)) - 1.0)
    C = y.shape[1]
    neg = jnp.full((1, C), _NEG, jnp.float32)
    ym1 = jnp.concatenate([neg, y[:L - 1, :]], axis=0)
    yp1 = jnp.concatenate([y[1:, :], neg], axis=0)
    pf = jnp.maximum(jnp.maximum(ym1, y), yp1)
    r_io = jax.lax.broadcasted_iota(jnp.int32, (256, 512), 0)
    c_io = jax.lax.broadcasted_iota(jnp.int32, (256, 512), 1)
    dsel = (c_io == 2 * r_io).astype(jnp.float32)
    for t in range(L // 512):
        o_ref[0, t * 256:(t + 1) * 256, :] = jnp.dot(
            dsel, pf[t * 512:(t + 1) * 512, :],
            preferred_element_type=jnp.float32)


def _conv_distil(x, w0, w1, w2, b, scale, bb):
    b_, L, C = x.shape
    return pl.pallas_call(
        functools.partial(_conv_distil_krn, L=L),
        grid=(b_,),
        in_specs=[
            pl.BlockSpec((1, L, C), lambda i: (i, 0, 0)),
            pl.BlockSpec((C, C), lambda i: (0, 0)),
            pl.BlockSpec((C, C), lambda i: (0, 0)),
            pl.BlockSpec((C, C), lambda i: (0, 0)),
            pl.BlockSpec((1, C), lambda i: (0, 0)),
            pl.BlockSpec((1, C), lambda i: (0, 0)),
            pl.BlockSpec((1, C), lambda i: (0, 0)),
        ],
        out_specs=pl.BlockSpec((1, L // 2, C), lambda i: (i, 0, 0)),
        out_shape=jax.ShapeDtypeStruct((b_, L // 2, C), jnp.float32),
    )(x, w0, w1, w2, b, scale, bb)


def _final_krn(x_ref, g_ref, be_ref, w_ref, b_ref, o_ref):
    y = _ln(x_ref[...], g_ref[...], be_ref[...])
    o_ref[...] = jnp.dot(y, w_ref[...],
                         preferred_element_type=jnp.float32) + b_ref[...]


def _final_proj(x, g, beta, w, b):
    M, K = x.shape
    N = w.shape[1]
    TM = 256
    return pl.pallas_call(
        _final_krn,
        grid=(M // TM,),
        in_specs=[
            pl.BlockSpec((TM, K), lambda i: (i, 0)),
            pl.BlockSpec((1, K), lambda i: (0, 0)),
            pl.BlockSpec((1, K), lambda i: (0, 0)),
            pl.BlockSpec((K, N), lambda i: (0, 0)),
            pl.BlockSpec((1, N), lambda i: (0, 0)),
        ],
        out_specs=pl.BlockSpec((TM, N), lambda i: (i, 0)),
        out_shape=jax.ShapeDtypeStruct((M, N), jnp.float32),
    )(x, g, beta, w, b)


def _row(v):
    return v.reshape(1, -1)


def _build_xc(x, mark, conv_w, temp_w):
    xc = jnp.concatenate(
        [jnp.roll(x, 1, axis=1), x, jnp.roll(x, -1, axis=1), mark], axis=-1)
    kin = xc.shape[-1]
    pad = (-kin) % 32
    xc = jnp.pad(xc, ((0, 0), (0, 0), (0, pad)))
    w = jnp.concatenate([conv_w[:, :, 0].T, conv_w[:, :, 1].T,
                         conv_w[:, :, 2].T, temp_w.T], axis=0)
    w = jnp.pad(w, ((0, pad), (0, 0)))
    return xc, w


def _qkv_cat(p, names):
    w = jnp.concatenate([p['w' + n].T for n in names], axis=1)
    b = jnp.concatenate([p['b' + n] for n in names])
    return w, _row(b)


def _u_of(LQ):
    return min(FACTOR * int(np.ceil(np.log(LQ))), LQ)


def kernel(batch_x, batch_x_mark, batch_y_mark, params):
    p = params
    pos = jnp.asarray(_POS_2048)

    e0 = p['enc0']
    xc, we = _build_xc(batch_x, batch_x_mark,
                       p['enc_emb']['conv_w'], p['enc_emb']['temp_w'])
    wqkv, bqkv = _qkv_cat(e0['attn'], ['q', 'k', 'v'])
    emb, q, k, v = _embed_qkv(xc, we, pos, wqkv, bqkv)

    ctx = _fused_attn(q, k, v, jnp.asarray(_CNTS[0]), _u_of(SEQ), False)
    x2 = _proj_ffn(ctx.reshape(-1, D_MODEL), emb.reshape(-1, D_MODEL), e0['attn']['wo'].T,
                   _row(e0['attn']['bo']), _row(e0['ln1_g']), _row(e0['ln1_b']),
                   e0['w1'].T, _row(e0['b1']), e0['w2'].T, _row(e0['b2']),
                   _row(e0['ln2_g']), _row(e0['ln2_b']), B, SEQ)

    c0 = p['conv0']
    scale = _row(c0['bn_g'] * (1.0 / np.sqrt(1.0 + 1e-5)))
    x = _conv_distil(x2.reshape(B, SEQ, D_MODEL), c0['w'][:, :, 0].T,
                     c0['w'][:, :, 1].T, c0['w'][:, :, 2].T, _row(c0['b']),
                     scale, _row(c0['bn_b']))

    e1 = p['enc1']
    L1 = SEQ // 2
    wqkv, bqkv = _qkv_cat(e1['attn'], ['q', 'k', 'v'])
    q, k, v = _qkv_proj(x, wqkv, bqkv, 3)
    ctx = _fused_attn(q, k, v, jnp.asarray(_CNTS[1]), _u_of(L1), False)
    x2 = _proj_ffn(ctx.reshape(-1, D_MODEL), x.reshape(-1, D_MODEL), e1['attn']['wo'].T,
                   _row(e1['attn']['bo']), _row(e1['ln1_g']), _row(e1['ln1_b']),
                   e1['w1'].T, _row(e1['b1']), e1['w2'].T, _row(e1['b2']),
                   _row(e1['ln2_g']), _row(e1['ln2_b']), B, L1,
                   _row(p['enc_norm_g']), _row(p['enc_norm_b']))
    enc_out = x2.reshape(B, L1, D_MODEL)

    d0 = p['dec0']
    L_DEC = LABEL_LEN + PRED_LEN
    dec_inp = jnp.concatenate(
        [batch_x[:, -LABEL_LEN:, :],
         jnp.zeros((B, PRED_LEN, ENC_IN), jnp.float32)], axis=1)
    xc, we = _build_xc(dec_inp, batch_y_mark,
                       p['dec_emb']['conv_w'], p['dec_emb']['temp_w'])
    wqkv, bqkv = _qkv_cat(d0['self'], ['q', 'k', 'v'])
    emb, q, k, v = _embed_qkv(xc, we, pos, wqkv, bqkv)

    ctx = _fused_attn(q, k, v, jnp.asarray(_CNTS[2]), _u_of(L_DEC), True)
    t = _proj_res_ln(ctx.reshape(-1, D_MODEL), emb.reshape(-1, D_MODEL), d0['self']['wo'].T,
                     _row(d0['self']['bo']), _row(d0['ln1_g']),
                     _row(d0['ln1_b']), B, L_DEC)

    q2, = _qkv_proj(t.reshape(B, L_DEC, D_MODEL), d0['cross']['wq'].T,
                    _row(d0['cross']['bq']), 1)
    wkv, bkv = _qkv_cat(d0['cross'], ['k', 'v'])
    k2, v2 = _qkv_proj(enc_out, wkv, bkv, 2)
    ctx2 = _fused_attn(q2, k2, v2, jnp.asarray(_CNTS[3]), _u_of(L_DEC), False)
    x2 = _proj_ffn(ctx2.reshape(-1, D_MODEL), t, d0['cross']['wo'].T, _row(d0['cross']['bo']),
                   _row(d0['ln2_g']), _row(d0['ln2_b']),
                   d0['w1'].T, _row(d0['b1']), d0['w2'].T, _row(d0['b2']),
                   _row(d0['ln3_g']), _row(d0['ln3_b']), B, L_DEC)

    xdec = x2.reshape(B, L_DEC, D_MODEL)[:, -PRED_LEN:, :]
    out = _final_proj(xdec.reshape(B * PRED_LEN, D_MODEL),
                      _row(p['dec_norm_g']), _row(p['dec_norm_b']),
                      p['proj_w'].T, _row(p['proj_b']))
    return out.reshape(B, PRED_LEN, C_OUT)


# speedup vs baseline: 6.3844x; 1.0868x over previous
import functools
import math

import jax
import jax.numpy as jnp
import numpy as np
from jax.experimental import pallas as pl
from jax.experimental.pallas import tpu as pltpu

B = 4
SEQ = 2048
ENC_IN = 7
C_OUT = 7
MARK_DIM = 4
D_MODEL = 256
N_HEADS = 8
HD = D_MODEL // N_HEADS
D_FF = 1024
FACTOR = 5
LABEL_LEN = 1024
PRED_LEN = 1024

_NEG = -1e30


def _pos_embed_np(L, d):
    pos = np.arange(L)[:, None].astype(np.float64)
    div = np.exp(np.arange(0, d, 2) * -(np.log(10000.0) / d))
    pe = np.zeros((L, d))
    pe[:, 0::2] = np.sin(pos * div)
    pe[:, 1::2] = np.cos(pos * div)
    return pe.astype(np.float32)


_POS_2048 = _pos_embed_np(SEQ, D_MODEL)


def _u_part(Lk):
    return min(FACTOR * int(np.ceil(np.log(Lk))), Lk)


def _tf_rotl(x, d):
    return ((x << np.uint32(d)) | (x >> np.uint32(32 - d))).astype(np.uint32)


def _tf2x32(k0, k1, x0, x1):
    x0 = x0.astype(np.uint32).copy()
    x1 = x1.astype(np.uint32).copy()
    ks0, ks1 = np.uint32(k0), np.uint32(k1)
    ks2 = np.uint32(ks0 ^ ks1 ^ np.uint32(0x1BD11BDA))
    rot = [13, 15, 26, 6, 17, 29, 16, 24]
    x0 = (x0 + ks0).astype(np.uint32)
    x1 = (x1 + ks1).astype(np.uint32)
    keys = [(ks1, ks2), (ks2, ks0), (ks0, ks1), (ks1, ks2), (ks2, ks0)]
    for r in range(5):
        for d in (rot[:4] if r % 2 == 0 else rot[4:]):
            x0 = (x0 + x1).astype(np.uint32)
            x1 = _tf_rotl(x1, d)
            x1 = (x1 ^ x0).astype(np.uint32)
        ka, kb = keys[r]
        x0 = (x0 + ka).astype(np.uint32)
        x1 = (x1 + kb + np.uint32(r + 1)).astype(np.uint32)
    return x0, x1


def _tf_counter(n):
    cnt = np.arange(n, dtype=np.uint64)
    return ((cnt >> np.uint64(32)).astype(np.uint32),
            (cnt & np.uint64(0xFFFFFFFF)).astype(np.uint32))


def _tf_key(seed):
    return np.uint32(np.uint64(seed) >> np.uint64(32)), np.uint32(seed & 0xFFFFFFFF)


def _tf_fold_in(key, data):
    d0, d1 = _tf_key(int(data))
    x0, x1 = _tf2x32(key[0], key[1], np.array([d0]), np.array([d1]))
    return np.uint32(x0[0]), np.uint32(x1[0])


def _tf_randint_pow2(key, shape, span):
    c0, c1 = _tf_counter(2)
    s0, s1 = _tf2x32(key[0], key[1], c0, c1)
    k2 = (s0[1], s1[1])
    n = int(np.prod(shape))
    c0, c1 = _tf_counter(n)
    b0, b1 = _tf2x32(k2[0], k2[1], c0, c1)
    return ((b0 ^ b1) % np.uint32(span)).astype(np.int64).reshape(shape)


def _cnt_matrix_np(fold, LQ, LK):
    U = _u_part(LK)
    key = _tf_fold_in(_tf_key(42), fold)
    idx = _tf_randint_pow2(key, (LQ, U), LK)
    cnt = np.zeros((LQ, LK), np.int8)
    np.add.at(cnt, (np.arange(LQ)[:, None], idx), 1)
    return cnt


_CNTS = {
    0: _cnt_matrix_np(0, SEQ, SEQ),
    1: _cnt_matrix_np(1, SEQ // 2, SEQ // 2),
    2: _cnt_matrix_np(2, SEQ, SEQ),
    3: _cnt_matrix_np(3, SEQ, SEQ // 2),
}


def _qkv_krn(x_ref, w_ref, b_ref, *o_refs):
    y = jnp.dot(x_ref[0], w_ref[...],
                preferred_element_type=jnp.float32) + b_ref[...]
    for t, o_ref in enumerate(o_refs):
        o_ref[0] = y[:, t * D_MODEL:(t + 1) * D_MODEL]


def _qkv_proj(x, w, b, n_out):
    b_, L, K = x.shape
    TL = 512
    outs = pl.pallas_call(
        _qkv_krn,
        grid=(b_, L // TL),
        in_specs=[
            pl.BlockSpec((1, TL, K), lambda i, j: (i, j, 0)),
            pl.BlockSpec((K, n_out * D_MODEL), lambda i, j: (0, 0)),
            pl.BlockSpec((1, n_out * D_MODEL), lambda i, j: (0, 0)),
        ],
        out_specs=[pl.BlockSpec((1, TL, D_MODEL), lambda i, j: (i, j, 0))
                   for _ in range(n_out)],
        out_shape=[jax.ShapeDtypeStruct((b_, L, D_MODEL), jnp.float32)
                   for _ in range(n_out)],
    )(x, w, b)
    return outs


def _embed_qkv_krn(xc_ref, we_ref, pos_ref, w_ref, b_ref, e_ref, *o_refs):
    emb = jnp.dot(xc_ref[0], we_ref[...],
                  preferred_element_type=jnp.float32) + pos_ref[...]
    e_ref[0] = emb
    y = jnp.dot(emb, w_ref[...],
                preferred_element_type=jnp.float32) + b_ref[...]
    for t, o_ref in enumerate(o_refs):
        o_ref[0] = y[:, t * D_MODEL:(t + 1) * D_MODEL]


def _embed_qkv(xc, we, pos, w, b):
    b_, L, kin = xc.shape
    TL = 512
    emb, q, k, v = pl.pallas_call(
        _embed_qkv_krn,
        grid=(b_, L // TL),
        in_specs=[
            pl.BlockSpec((1, TL, kin), lambda i, j: (i, j, 0)),
            pl.BlockSpec((kin, D_MODEL), lambda i, j: (0, 0)),
            pl.BlockSpec((TL, D_MODEL), lambda i, j: (j, 0)),
            pl.BlockSpec((D_MODEL, 3 * D_MODEL), lambda i, j: (0, 0)),
            pl.BlockSpec((1, 3 * D_MODEL), lambda i, j: (0, 0)),
        ],
        out_specs=[pl.BlockSpec((1, TL, D_MODEL), lambda i, j: (i, j, 0))
                   for _ in range(4)],
        out_shape=[jax.ShapeDtypeStruct((b_, L, D_MODEL), jnp.float32)
                   for _ in range(4)],
    )(xc, we, pos, w, b)
    return emb, q, k, v


def _ln(y, g, b):
    m = jnp.mean(y, axis=-1, keepdims=True)
    v = jnp.mean((y - m) ** 2, axis=-1, keepdims=True)
    return (y - m) * jax.lax.rsqrt(v + 1e-5) * g + b


def _proj_res_ln_krn(x_ref, r_ref, w_ref, b_ref, g_ref, be_ref, o_ref):
    y = jnp.dot(x_ref[...], w_ref[...],
                preferred_element_type=jnp.float32) + b_ref[...] + r_ref[...]
    o_ref[...] = _ln(y, g_ref[...], be_ref[...])


def _proj_res_ln(ctx, resid, w, b, g, beta, b_, L):
    N = w.shape[1]
    TL = 512
    nl = L // TL
    return pl.pallas_call(
        _proj_res_ln_krn,
        grid=(b_, nl),
        in_specs=[
            pl.BlockSpec((TL, N), lambda i, j: (i * nl + j, 0)),
            pl.BlockSpec((TL, N), lambda i, j: (i * nl + j, 0)),
            pl.BlockSpec((D_MODEL, N), lambda i, j: (0, 0)),
            pl.BlockSpec((1, N), lambda i, j: (0, 0)),
            pl.BlockSpec((1, N), lambda i, j: (0, 0)),
            pl.BlockSpec((1, N), lambda i, j: (0, 0)),
        ],
        out_specs=pl.BlockSpec((TL, N), lambda i, j: (i * nl + j, 0)),
        out_shape=jax.ShapeDtypeStruct((b_ * L, N), jnp.float32),
    )(ctx, resid, w, b, g, beta)


def _gelu_exact(x):
    return x * 0.5 * (1.0 + jax.lax.erf(x * (1.0 / math.sqrt(2.0))))


def _proj_ffn_krn(x_ref, r_ref, wo_ref, bo_ref, g1_ref, be1_ref,
                  w1_ref, b1_ref, w2_ref, b2_ref, g2_ref, be2_ref,
                  g3_ref, be3_ref, o_ref, *, three_ln):
    y = jnp.dot(x_ref[...], wo_ref[...],
                preferred_element_type=jnp.float32) + bo_ref[...] + r_ref[...]
    y = _ln(y, g1_ref[...], be1_ref[...])
    h = _gelu_exact(jnp.dot(y, w1_ref[...],
                            preferred_element_type=jnp.float32) + b1_ref[...])
    z = y + jnp.dot(h, w2_ref[...],
                    preferred_element_type=jnp.float32) + b2_ref[...]
    z = _ln(z, g2_ref[...], be2_ref[...])
    if three_ln:
        z = _ln(z, g3_ref[...], be3_ref[...])
    o_ref[...] = z


def _proj_ffn(ctx, resid, wo, bo, g1, be1, w1, b1, w2, b2, g2, be2,
              b_, L, g3=None, be3=None):
    three_ln = g3 is not None
    if g3 is None:
        g3, be3 = g2, be2
    TL = 512
    nl = L // TL
    return pl.pallas_call(
        functools.partial(_proj_ffn_krn, three_ln=three_ln),
        grid=(b_, nl),
        in_specs=[
            pl.BlockSpec((TL, D_MODEL), lambda i, j: (i * nl + j, 0)),
            pl.BlockSpec((TL, D_MODEL), lambda i, j: (i * nl + j, 0)),
            pl.BlockSpec((D_MODEL, D_MODEL), lambda i, j: (0, 0)),
            pl.BlockSpec((1, D_MODEL), lambda i, j: (0, 0)),
            pl.BlockSpec((1, D_MODEL), lambda i, j: (0, 0)),
            pl.BlockSpec((1, D_MODEL), lambda i, j: (0, 0)),
            pl.BlockSpec((D_MODEL, D_FF), lambda i, j: (0, 0)),
            pl.BlockSpec((1, D_FF), lambda i, j: (0, 0)),
            pl.BlockSpec((D_FF, D_MODEL), lambda i, j: (0, 0)),
            pl.BlockSpec((1, D_MODEL), lambda i, j: (0, 0)),
            pl.BlockSpec((1, D_MODEL), lambda i, j: (0, 0)),
            pl.BlockSpec((1, D_MODEL), lambda i, j: (0, 0)),
            pl.BlockSpec((1, D_MODEL), lambda i, j: (0, 0)),
            pl.BlockSpec((1, D_MODEL), lambda i, j: (0, 0)),
        ],
        out_specs=pl.BlockSpec((TL, D_MODEL), lambda i, j: (i * nl + j, 0)),
        out_shape=jax.ShapeDtypeStruct((b_ * L, D_MODEL), jnp.float32),
    )(ctx, resid, wo, bo, g1, be1, w1, b1, w2, b2, g2, be2, g3, be3)


def _fattn_krn(q_ref, k_ref, v_ref, c_ref, o_ref, mc_ref, p_ref,
               cm_ref, *, u, upad, LQ, LK, masked, HG):
    TQ = 256

    kb = k_ref[0]
    for t in range(LQ // TQ):
        sl = slice(t * TQ, (t + 1) * TQ)
        cf = c_ref[sl, :].astype(jnp.float32)
        madd = jnp.where(cf > 0.5, 0.0, _NEG)
        sk = jnp.dot(cf, kb, preferred_element_type=jnp.float32)
        for h in range(HG):
            hs = slice(h * HD, (h + 1) * HD)
            qt = q_ref[0, sl, hs]
            s = jax.lax.dot_general(qt, kb[:, hs], (((1,), (1,)), ((), ())),
                                    preferred_element_type=jnp.float32)
            mxc = jnp.max(s + madd, axis=1, keepdims=True)
            smc = jnp.sum(qt * sk[:, hs], axis=1,
                          keepdims=True) * (1.0 / LK)
            mc_ref[sl, h:h + 1] = mxc - smc

    m = jnp.transpose(mc_ref[...])
    lane_q = jax.lax.broadcasted_iota(jnp.int32, (HG, LQ), 1)
    if masked:
        lane_k = jax.lax.broadcasted_iota(jnp.int32, (HG, LK), 1)
    for i in range(u):
        mx = jnp.max(m, axis=-1, keepdims=True)
        am = jnp.min(jnp.where(m == mx, lane_q, LQ), axis=-1, keepdims=True)
        p_ref[:, i, :] = (lane_q == am).astype(jnp.float32)
        if masked:
            cm_ref[:, i, :] = (lane_k > am).astype(jnp.float32)
        m = jnp.where(lane_q == am, _NEG, m)
    for i in range(u, upad):
        p_ref[:, i, :] = jnp.zeros((HG, LQ), jnp.float32)
        if masked:
            cm_ref[:, i, :] = jnp.zeros((HG, LK), jnp.float32)

    ones = jnp.ones((upad, HD), jnp.float32)
    if masked:
        TT = 256
        r_io = jax.lax.broadcasted_iota(jnp.int32, (TT, TT), 0)
        c_io = jax.lax.broadcasted_iota(jnp.int32, (TT, TT), 1)
        tri = (r_io >= c_io).astype(jnp.float32)
    for h in range(HG):
        hs = slice(h * HD, (h + 1) * HD)
        V = v_ref[0, :, hs]
        ph = p_ref[h]
        qred = jnp.dot(ph, q_ref[0, :, hs],
                       preferred_element_type=jnp.float32)
        s = jax.lax.dot_general(qred, kb[:, hs], (((1,), (1,)), ((), ())),
                                preferred_element_type=jnp.float32)
        s = s * (1.0 / math.sqrt(HD))
        if masked:
            s = s + cm_ref[h] * _NEG
        mx = jnp.max(s, axis=-1, keepdims=True)
        e = jnp.exp(s - mx)
        p = e / jnp.sum(e, axis=-1, keepdims=True)
        upd = jnp.dot(p, V, preferred_element_type=jnp.float32)
        ctxsel = jax.lax.dot_general(ph, upd, (((0,), (0,)), ((), ())),
                                     preferred_element_type=jnp.float32)
        covm = jax.lax.dot_general(ph, ones, (((0,), (0,)), ((), ())),
                                   preferred_element_type=jnp.float32)
        if masked:
            carry = jnp.zeros((1, HD), jnp.float32)
            for t in range(LK // TT):
                vt = V[t * TT:(t + 1) * TT, :]
                cum = jnp.dot(tri, vt,
                              preferred_element_type=jnp.float32) + carry
                sl = slice(t * TT, (t + 1) * TT)
                o_ref[0, sl, hs] = (cum * (1.0 - covm[sl, :])
                                    + ctxsel[sl, :])
                carry = cum[TT - 1:TT, :]
        else:
            mean = jnp.sum(V, axis=0, keepdims=True) * (1.0 / LK)
            base = jnp.broadcast_to(mean, (LQ, HD))
            o_ref[0, :, hs] = base * (1.0 - covm) + ctxsel


def _fused_attn(q, k, v, cnt, u, masked):
    b, LQ, _ = q.shape
    LK = k.shape[1]
    upad = ((u + 7) // 8) * 8
    HG = 4
    W = HG * HD
    scratch = [
        pltpu.VMEM((LQ, HG), jnp.float32),
        pltpu.VMEM((HG, upad, LQ), jnp.float32),
        pltpu.VMEM((HG, upad, LK), jnp.float32),
    ]
    return pl.pallas_call(
        functools.partial(_fattn_krn, u=u, upad=upad, LQ=LQ, LK=LK,
                          masked=masked, HG=HG),
        grid=(b, N_HEADS // HG),
        in_specs=[
            pl.BlockSpec((1, LQ, W), lambda g, j: (g, 0, j)),
            pl.BlockSpec((1, LK, W), lambda g, j: (g, 0, j)),
            pl.BlockSpec((1, LK, W), lambda g, j: (g, 0, j)),
            pl.BlockSpec((LQ, LK), lambda g, j: (0, 0)),
        ],
        out_specs=pl.BlockSpec((1, LQ, W), lambda g, j: (g, 0, j)),
        out_shape=jax.ShapeDtypeStruct((b, LQ, D_MODEL), jnp.float32),
        scratch_shapes=scratch,
    )(q, k, v, cnt)


def _conv_distil_krn(x_ref, w0_ref, w1_ref, w2_ref, b_ref, sc_ref, bb_ref,
                     o_ref, *, L):
    x = x_ref[0]
    xm1 = jnp.concatenate([x[L - 1:L, :], x[:L - 1, :]], axis=0)
    xp1 = jnp.concatenate([x[1:, :], x[:1, :]], axis=0)
    y = (jnp.dot(xm1, w0_ref[...], preferred_element_type=jnp.float32)
         + jnp.dot(x, w1_ref[...], preferred_element_type=jnp.float32)
         + jnp.dot(xp1, w2_ref[...], preferred_element_type=jnp.float32)
         + b_ref[...])
    y = y * sc_ref[...] + bb_ref[...]
    y = jnp.where(y > 0, y, jnp.exp(jnp.minimum(y, 0.0)) - 1.0)
    C = y.shape[1]
    neg = jnp.full((1, C), _NEG, jnp.float32)
    ym1 = jnp.concatenate([neg, y[:L - 1, :]], axis=0)
    yp1 = jnp.concatenate([y[1:, :], neg], axis=0)
    pf = jnp.maximum(jnp.maximum(ym1, y), yp1)
    r_io = jax.lax.broadcasted_iota(jnp.int32, (256, 512), 0)
    c_io = jax.lax.broadcasted_iota(jnp.int32, (256, 512), 1)
    dsel = (c_io == 2 * r_io).astype(jnp.float32)
    for t in range(L // 512):
        o_ref[0, t * 256:(t + 1) * 256, :] = jnp.dot(
            dsel, pf[t * 512:(t + 1) * 512, :],
            preferred_element_type=jnp.float32)


def _conv_distil(x, w0, w1, w2, b, scale, bb):
    b_, L, C = x.shape
    return pl.pallas_call(
        functools.partial(_conv_distil_krn, L=L),
        grid=(b_,),
        in_specs=[
            pl.BlockSpec((1, L, C), lambda i: (i, 0, 0)),
            pl.BlockSpec((C, C), lambda i: (0, 0)),
            pl.BlockSpec((C, C), lambda i: (0, 0)),
            pl.BlockSpec((C, C), lambda i: (0, 0)),
            pl.BlockSpec((1, C), lambda i: (0, 0)),
            pl.BlockSpec((1, C), lambda i: (0, 0)),
            pl.BlockSpec((1, C), lambda i: (0, 0)),
        ],
        out_specs=pl.BlockSpec((1, L // 2, C), lambda i: (i, 0, 0)),
        out_shape=jax.ShapeDtypeStruct((b_, L // 2, C), jnp.float32),
    )(x, w0, w1, w2, b, scale, bb)


def _final_krn(x_ref, g_ref, be_ref, w_ref, b_ref, o_ref):
    y = _ln(x_ref[...], g_ref[...], be_ref[...])
    o_ref[...] = jnp.dot(y, w_ref[...],
                         preferred_element_type=jnp.float32) + b_ref[...]


def _final_proj(x, g, beta, w, b):
    M, K = x.shape
    N = w.shape[1]
    TM = 512
    return pl.pallas_call(
        _final_krn,
        grid=(M // TM,),
        in_specs=[
            pl.BlockSpec((TM, K), lambda i: (i, 0)),
            pl.BlockSpec((1, K), lambda i: (0, 0)),
            pl.BlockSpec((1, K), lambda i: (0, 0)),
            pl.BlockSpec((K, N), lambda i: (0, 0)),
            pl.BlockSpec((1, N), lambda i: (0, 0)),
        ],
        out_specs=pl.BlockSpec((TM, N), lambda i: (i, 0)),
        out_shape=jax.ShapeDtypeStruct((M, N), jnp.float32),
    )(x, g, beta, w, b)


def _row(v):
    return v.reshape(1, -1)


def _build_xc(x, mark, conv_w, temp_w):
    xc = jnp.concatenate(
        [jnp.roll(x, 1, axis=1), x, jnp.roll(x, -1, axis=1), mark], axis=-1)
    kin = xc.shape[-1]
    pad = (-kin) % 32
    xc = jnp.pad(xc, ((0, 0), (0, 0), (0, pad)))
    w = jnp.concatenate([conv_w[:, :, 0].T, conv_w[:, :, 1].T,
                         conv_w[:, :, 2].T, temp_w.T], axis=0)
    w = jnp.pad(w, ((0, pad), (0, 0)))
    return xc, w


def _qkv_cat(p, names):
    w = jnp.concatenate([p['w' + n].T for n in names], axis=1)
    b = jnp.concatenate([p['b' + n] for n in names])
    return w, _row(b)


def _u_of(LQ):
    return min(FACTOR * int(np.ceil(np.log(LQ))), LQ)


def kernel(batch_x, batch_x_mark, batch_y_mark, params):
    p = params
    pos = jnp.asarray(_POS_2048)

    e0 = p['enc0']
    xc, we = _build_xc(batch_x, batch_x_mark,
                       p['enc_emb']['conv_w'], p['enc_emb']['temp_w'])
    wqkv, bqkv = _qkv_cat(e0['attn'], ['q', 'k', 'v'])
    emb, q, k, v = _embed_qkv(xc, we, pos, wqkv, bqkv)

    ctx = _fused_attn(q, k, v, jnp.asarray(_CNTS[0]), _u_of(SEQ), False)
    x2 = _proj_ffn(ctx.reshape(-1, D_MODEL), emb.reshape(-1, D_MODEL), e0['attn']['wo'].T,
                   _row(e0['attn']['bo']), _row(e0['ln1_g']), _row(e0['ln1_b']),
                   e0['w1'].T, _row(e0['b1']), e0['w2'].T, _row(e0['b2']),
                   _row(e0['ln2_g']), _row(e0['ln2_b']), B, SEQ)

    c0 = p['conv0']
    scale = _row(c0['bn_g'] * (1.0 / np.sqrt(1.0 + 1e-5)))
    x = _conv_distil(x2.reshape(B, SEQ, D_MODEL), c0['w'][:, :, 0].T,
                     c0['w'][:, :, 1].T, c0['w'][:, :, 2].T, _row(c0['b']),
                     scale, _row(c0['bn_b']))

    e1 = p['enc1']
    L1 = SEQ // 2
    wqkv, bqkv = _qkv_cat(e1['attn'], ['q', 'k', 'v'])
    q, k, v = _qkv_proj(x, wqkv, bqkv, 3)
    ctx = _fused_attn(q, k, v, jnp.asarray(_CNTS[1]), _u_of(L1), False)
    x2 = _proj_ffn(ctx.reshape(-1, D_MODEL), x.reshape(-1, D_MODEL), e1['attn']['wo'].T,
                   _row(e1['attn']['bo']), _row(e1['ln1_g']), _row(e1['ln1_b']),
                   e1['w1'].T, _row(e1['b1']), e1['w2'].T, _row(e1['b2']),
                   _row(e1['ln2_g']), _row(e1['ln2_b']), B, L1,
                   _row(p['enc_norm_g']), _row(p['enc_norm_b']))
    enc_out = x2.reshape(B, L1, D_MODEL)

    d0 = p['dec0']
    L_DEC = LABEL_LEN + PRED_LEN
    dec_inp = jnp.concatenate(
        [batch_x[:, -LABEL_LEN:, :],
         jnp.zeros((B, PRED_LEN, ENC_IN), jnp.float32)], axis=1)
    xc, we = _build_xc(dec_inp, batch_y_mark,
                       p['dec_emb']['conv_w'], p['dec_emb']['temp_w'])
    wqkv, bqkv = _qkv_cat(d0['self'], ['q', 'k', 'v'])
    emb, q, k, v = _embed_qkv(xc, we, pos, wqkv, bqkv)

    ctx = _fused_attn(q, k, v, jnp.asarray(_CNTS[2]), _u_of(L_DEC), True)
    t = _proj_res_ln(ctx.reshape(-1, D_MODEL), emb.reshape(-1, D_MODEL), d0['self']['wo'].T,
                     _row(d0['self']['bo']), _row(d0['ln1_g']),
                     _row(d0['ln1_b']), B, L_DEC)

    q2, = _qkv_proj(t.reshape(B, L_DEC, D_MODEL), d0['cross']['wq'].T,
                    _row(d0['cross']['bq']), 1)
    wkv, bkv = _qkv_cat(d0['cross'], ['k', 'v'])
    k2, v2 = _qkv_proj(enc_out, wkv, bkv, 2)
    ctx2 = _fused_attn(q2, k2, v2, jnp.asarray(_CNTS[3]), _u_of(L_DEC), False)
    x2 = _proj_ffn(ctx2.reshape(-1, D_MODEL), t, d0['cross']['wo'].T, _row(d0['cross']['bo']),
                   _row(d0['ln2_g']), _row(d0['ln2_b']),
                   d0['w1'].T, _row(d0['b1']), d0['w2'].T, _row(d0['b2']),
                   _row(d0['ln3_g']), _row(d0['ln3_b']), B, L_DEC)

    xdec = x2.reshape(B, L_DEC, D_MODEL)[:, -PRED_LEN:, :]
    out = _final_proj(xdec.reshape(B * PRED_LEN, D_MODEL),
                      _row(p['dec_norm_g']), _row(p['dec_norm_b']),
                      p['proj_w'].T, _row(p['proj_b']))
    return out.reshape(B, PRED_LEN, C_OUT)


# speedup vs baseline: 7.2479x; 1.1353x over previous
import functools
import math

import jax
import jax.numpy as jnp
import numpy as np
from jax.experimental import pallas as pl
from jax.experimental.pallas import tpu as pltpu

B = 4
SEQ = 2048
ENC_IN = 7
C_OUT = 7
MARK_DIM = 4
D_MODEL = 256
N_HEADS = 8
HD = D_MODEL // N_HEADS
D_FF = 1024
FACTOR = 5
LABEL_LEN = 1024
PRED_LEN = 1024

_NEG = -1e30


def _pos_embed_np(L, d):
    pos = np.arange(L)[:, None].astype(np.float64)
    div = np.exp(np.arange(0, d, 2) * -(np.log(10000.0) / d))
    pe = np.zeros((L, d))
    pe[:, 0::2] = np.sin(pos * div)
    pe[:, 1::2] = np.cos(pos * div)
    return pe.astype(np.float32)


_POS_2048 = _pos_embed_np(SEQ, D_MODEL)


def _u_part(Lk):
    return min(FACTOR * int(np.ceil(np.log(Lk))), Lk)


def _tf_rotl(x, d):
    return ((x << np.uint32(d)) | (x >> np.uint32(32 - d))).astype(np.uint32)


def _tf2x32(k0, k1, x0, x1):
    x0 = x0.astype(np.uint32).copy()
    x1 = x1.astype(np.uint32).copy()
    ks0, ks1 = np.uint32(k0), np.uint32(k1)
    ks2 = np.uint32(ks0 ^ ks1 ^ np.uint32(0x1BD11BDA))
    rot = [13, 15, 26, 6, 17, 29, 16, 24]
    x0 = (x0 + ks0).astype(np.uint32)
    x1 = (x1 + ks1).astype(np.uint32)
    keys = [(ks1, ks2), (ks2, ks0), (ks0, ks1), (ks1, ks2), (ks2, ks0)]
    for r in range(5):
        for d in (rot[:4] if r % 2 == 0 else rot[4:]):
            x0 = (x0 + x1).astype(np.uint32)
            x1 = _tf_rotl(x1, d)
            x1 = (x1 ^ x0).astype(np.uint32)
        ka, kb = keys[r]
        x0 = (x0 + ka).astype(np.uint32)
        x1 = (x1 + kb + np.uint32(r + 1)).astype(np.uint32)
    return x0, x1


def _tf_counter(n):
    cnt = np.arange(n, dtype=np.uint64)
    return ((cnt >> np.uint64(32)).astype(np.uint32),
            (cnt & np.uint64(0xFFFFFFFF)).astype(np.uint32))


def _tf_key(seed):
    return np.uint32(np.uint64(seed) >> np.uint64(32)), np.uint32(seed & 0xFFFFFFFF)


def _tf_fold_in(key, data):
    d0, d1 = _tf_key(int(data))
    x0, x1 = _tf2x32(key[0], key[1], np.array([d0]), np.array([d1]))
    return np.uint32(x0[0]), np.uint32(x1[0])


def _tf_randint_pow2(key, shape, span):
    c0, c1 = _tf_counter(2)
    s0, s1 = _tf2x32(key[0], key[1], c0, c1)
    k2 = (s0[1], s1[1])
    n = int(np.prod(shape))
    c0, c1 = _tf_counter(n)
    b0, b1 = _tf2x32(k2[0], k2[1], c0, c1)
    return ((b0 ^ b1) % np.uint32(span)).astype(np.int64).reshape(shape)


def _cnt_matrix_np(fold, LQ, LK):
    U = _u_part(LK)
    key = _tf_fold_in(_tf_key(42), fold)
    idx = _tf_randint_pow2(key, (LQ, U), LK)
    cnt = np.zeros((LQ, LK), np.int8)
    np.add.at(cnt, (np.arange(LQ)[:, None], idx), 1)
    return cnt


_CNTS = {
    0: _cnt_matrix_np(0, SEQ, SEQ),
    1: _cnt_matrix_np(1, SEQ // 2, SEQ // 2),
    2: _cnt_matrix_np(2, SEQ, SEQ),
    3: _cnt_matrix_np(3, SEQ, SEQ // 2),
}


def _qkv_krn(x_ref, w_ref, b_ref, *o_refs):
    y = jnp.dot(x_ref[0], w_ref[...],
                preferred_element_type=jnp.float32) + b_ref[...]
    for t, o_ref in enumerate(o_refs):
        o_ref[0] = y[:, t * D_MODEL:(t + 1) * D_MODEL]


def _qkv_proj(x, w, b, n_out):
    b_, L, K = x.shape
    TL = 512
    outs = pl.pallas_call(
        _qkv_krn,
        grid=(b_, L // TL),
        in_specs=[
            pl.BlockSpec((1, TL, K), lambda i, j: (i, j, 0)),
            pl.BlockSpec((K, n_out * D_MODEL), lambda i, j: (0, 0)),
            pl.BlockSpec((1, n_out * D_MODEL), lambda i, j: (0, 0)),
        ],
        out_specs=[pl.BlockSpec((1, TL, D_MODEL), lambda i, j: (i, j, 0))
                   for _ in range(n_out)],
        out_shape=[jax.ShapeDtypeStruct((b_, L, D_MODEL), jnp.float32)
                   for _ in range(n_out)],
    )(x, w, b)
    return outs


def _embed_qkv_krn(xc_ref, we_ref, pos_ref, w_ref, b_ref, e_ref, *o_refs):
    emb = jnp.dot(xc_ref[0], we_ref[...],
                  preferred_element_type=jnp.float32) + pos_ref[...]
    e_ref[0] = emb
    y = jnp.dot(emb, w_ref[...],
                preferred_element_type=jnp.float32) + b_ref[...]
    for t, o_ref in enumerate(o_refs):
        o_ref[0] = y[:, t * D_MODEL:(t + 1) * D_MODEL]


def _embed_qkv(xc, we, pos, w, b):
    b_, L, kin = xc.shape
    TL = 512
    emb, q, k, v = pl.pallas_call(
        _embed_qkv_krn,
        grid=(b_, L // TL),
        in_specs=[
            pl.BlockSpec((1, TL, kin), lambda i, j: (i, j, 0)),
            pl.BlockSpec((kin, D_MODEL), lambda i, j: (0, 0)),
            pl.BlockSpec((TL, D_MODEL), lambda i, j: (j, 0)),
            pl.BlockSpec((D_MODEL, 3 * D_MODEL), lambda i, j: (0, 0)),
            pl.BlockSpec((1, 3 * D_MODEL), lambda i, j: (0, 0)),
        ],
        out_specs=[pl.BlockSpec((1, TL, D_MODEL), lambda i, j: (i, j, 0))
                   for _ in range(4)],
        out_shape=[jax.ShapeDtypeStruct((b_, L, D_MODEL), jnp.float32)
                   for _ in range(4)],
    )(xc, we, pos, w, b)
    return emb, q, k, v


def _ln(y, g, b):
    m = jnp.mean(y, axis=-1, keepdims=True)
    v = jnp.mean((y - m) ** 2, axis=-1, keepdims=True)
    return (y - m) * jax.lax.rsqrt(v + 1e-5) * g + b


def _proj_res_ln_krn(x_ref, r_ref, w_ref, b_ref, g_ref, be_ref, o_ref):
    y = jnp.dot(x_ref[...], w_ref[...],
                preferred_element_type=jnp.float32) + b_ref[...] + r_ref[...]
    o_ref[...] = _ln(y, g_ref[...], be_ref[...])


def _proj_res_ln(ctx, resid, w, b, g, beta, b_, L):
    N = w.shape[1]
    TL = 512
    nl = L // TL
    return pl.pallas_call(
        _proj_res_ln_krn,
        grid=(b_, nl),
        in_specs=[
            pl.BlockSpec((TL, N), lambda i, j: (i * nl + j, 0)),
            pl.BlockSpec((TL, N), lambda i, j: (i * nl + j, 0)),
            pl.BlockSpec((D_MODEL, N), lambda i, j: (0, 0)),
            pl.BlockSpec((1, N), lambda i, j: (0, 0)),
            pl.BlockSpec((1, N), lambda i, j: (0, 0)),
            pl.BlockSpec((1, N), lambda i, j: (0, 0)),
        ],
        out_specs=pl.BlockSpec((TL, N), lambda i, j: (i * nl + j, 0)),
        out_shape=jax.ShapeDtypeStruct((b_ * L, N), jnp.float32),
    )(ctx, resid, w, b, g, beta)


def _gelu_exact(x):
    return x * 0.5 * (1.0 + jax.lax.erf(x * (1.0 / math.sqrt(2.0))))


def _proj_ffn_krn(x_ref, r_ref, wo_ref, bo_ref, g1_ref, be1_ref,
                  w1_ref, b1_ref, w2_ref, b2_ref, g2_ref, be2_ref,
                  g3_ref, be3_ref, o_ref, *, three_ln):
    y = jnp.dot(x_ref[...], wo_ref[...],
                preferred_element_type=jnp.float32) + bo_ref[...] + r_ref[...]
    y = _ln(y, g1_ref[...], be1_ref[...])
    h = _gelu_exact(jnp.dot(y, w1_ref[...],
                            preferred_element_type=jnp.float32) + b1_ref[...])
    z = y + jnp.dot(h, w2_ref[...],
                    preferred_element_type=jnp.float32) + b2_ref[...]
    z = _ln(z, g2_ref[...], be2_ref[...])
    if three_ln:
        z = _ln(z, g3_ref[...], be3_ref[...])
    o_ref[...] = z


def _proj_ffn(ctx, resid, wo, bo, g1, be1, w1, b1, w2, b2, g2, be2,
              b_, L, g3=None, be3=None):
    three_ln = g3 is not None
    if g3 is None:
        g3, be3 = g2, be2
    TL = 512
    nl = L // TL
    return pl.pallas_call(
        functools.partial(_proj_ffn_krn, three_ln=three_ln),
        grid=(b_, nl),
        in_specs=[
            pl.BlockSpec((TL, D_MODEL), lambda i, j: (i * nl + j, 0)),
            pl.BlockSpec((TL, D_MODEL), lambda i, j: (i * nl + j, 0)),
            pl.BlockSpec((D_MODEL, D_MODEL), lambda i, j: (0, 0)),
            pl.BlockSpec((1, D_MODEL), lambda i, j: (0, 0)),
            pl.BlockSpec((1, D_MODEL), lambda i, j: (0, 0)),
            pl.BlockSpec((1, D_MODEL), lambda i, j: (0, 0)),
            pl.BlockSpec((D_MODEL, D_FF), lambda i, j: (0, 0)),
            pl.BlockSpec((1, D_FF), lambda i, j: (0, 0)),
            pl.BlockSpec((D_FF, D_MODEL), lambda i, j: (0, 0)),
            pl.BlockSpec((1, D_MODEL), lambda i, j: (0, 0)),
            pl.BlockSpec((1, D_MODEL), lambda i, j: (0, 0)),
            pl.BlockSpec((1, D_MODEL), lambda i, j: (0, 0)),
            pl.BlockSpec((1, D_MODEL), lambda i, j: (0, 0)),
            pl.BlockSpec((1, D_MODEL), lambda i, j: (0, 0)),
        ],
        out_specs=pl.BlockSpec((TL, D_MODEL), lambda i, j: (i * nl + j, 0)),
        out_shape=jax.ShapeDtypeStruct((b_ * L, D_MODEL), jnp.float32),
    )(ctx, resid, wo, bo, g1, be1, w1, b1, w2, b2, g2, be2, g3, be3)


def _fattn_krn(q_ref, k_ref, v_ref, c_ref, o_ref, mc_ref, p_ref,
               cm_ref, *, u, upad, LQ, LK, masked, HG):
    TQ = 256

    kb = k_ref[0]
    for t in range(LQ // TQ):
        sl = slice(t * TQ, (t + 1) * TQ)
        cf = c_ref[sl, :].astype(jnp.float32)
        madd = jnp.where(cf > 0.5, 0.0, _NEG)
        sk = jnp.dot(cf, kb, preferred_element_type=jnp.float32)
        for h in range(HG):
            hs = slice(h * HD, (h + 1) * HD)
            qt = q_ref[0, sl, hs]
            s = jax.lax.dot_general(qt, kb[:, hs], (((1,), (1,)), ((), ())),
                                    preferred_element_type=jnp.float32)
            mxc = jnp.max(s + madd, axis=1, keepdims=True)
            smc = jnp.sum(qt * sk[:, hs], axis=1,
                          keepdims=True) * (1.0 / LK)
            mc_ref[sl, h:h + 1] = mxc - smc

    m = jnp.transpose(mc_ref[...])
    lane_q = jax.lax.broadcasted_iota(jnp.int32, (HG, LQ), 1)
    if masked:
        lane_k = jax.lax.broadcasted_iota(jnp.int32, (HG, LK), 1)
    for i in range(u):
        mx = jnp.max(m, axis=-1, keepdims=True)
        am = jnp.min(jnp.where(m == mx, lane_q, LQ), axis=-1, keepdims=True)
        p_ref[:, i, :] = (lane_q == am).astype(jnp.float32)
        if masked:
            cm_ref[:, i, :] = (lane_k > am).astype(jnp.float32)
        m = jnp.where(lane_q == am, _NEG, m)
    for i in range(u, upad):
        p_ref[:, i, :] = jnp.zeros((HG, LQ), jnp.float32)
        if masked:
            cm_ref[:, i, :] = jnp.zeros((HG, LK), jnp.float32)

    ones = jnp.ones((upad, HD), jnp.float32)
    if masked:
        TT = 256
        r_io = jax.lax.broadcasted_iota(jnp.int32, (TT, TT), 0)
        c_io = jax.lax.broadcasted_iota(jnp.int32, (TT, TT), 1)
        tri = (r_io >= c_io).astype(jnp.float32)
    for h in range(HG):
        hs = slice(h * HD, (h + 1) * HD)
        V = v_ref[0, :, hs]
        ph = p_ref[h]
        qred = jnp.dot(ph, q_ref[0, :, hs],
                       preferred_element_type=jnp.float32)
        s = jax.lax.dot_general(qred, kb[:, hs], (((1,), (1,)), ((), ())),
                                preferred_element_type=jnp.float32)
        s = s * (1.0 / math.sqrt(HD))
        if masked:
            s = s + cm_ref[h] * _NEG
        mx = jnp.max(s, axis=-1, keepdims=True)
        e = jnp.exp(s - mx)
        p = e / jnp.sum(e, axis=-1, keepdims=True)
        upd = jnp.dot(p, V, preferred_element_type=jnp.float32)
        ctxsel = jax.lax.dot_general(ph, upd, (((0,), (0,)), ((), ())),
                                     preferred_element_type=jnp.float32)
        covm = jax.lax.dot_general(ph, ones, (((0,), (0,)), ((), ())),
                                   preferred_element_type=jnp.float32)
        if masked:
            carry = jnp.zeros((1, HD), jnp.float32)
            for t in range(LK // TT):
                vt = V[t * TT:(t + 1) * TT, :]
                cum = jnp.dot(tri, vt,
                              preferred_element_type=jnp.float32) + carry
                sl = slice(t * TT, (t + 1) * TT)
                o_ref[0, sl, hs] = (cum * (1.0 - covm[sl, :])
                                    + ctxsel[sl, :])
                carry = cum[TT - 1:TT, :]
        else:
            mean = jnp.sum(V, axis=0, keepdims=True) * (1.0 / LK)
            base = jnp.broadcast_to(mean, (LQ, HD))
            o_ref[0, :, hs] = base * (1.0 - covm) + ctxsel


def _fused_attn(q, k, v, cnt, u, masked):
    b, LQ, _ = q.shape
    LK = k.shape[1]
    upad = ((u + 7) // 8) * 8
    HG = 8
    W = HG * HD
    scratch = [
        pltpu.VMEM((LQ, HG), jnp.float32),
        pltpu.VMEM((HG, upad, LQ), jnp.float32),
        pltpu.VMEM((HG, upad, LK), jnp.float32),
    ]
    return pl.pallas_call(
        functools.partial(_fattn_krn, u=u, upad=upad, LQ=LQ, LK=LK,
                          masked=masked, HG=HG),
        grid=(b, N_HEADS // HG),
        in_specs=[
            pl.BlockSpec((1, LQ, W), lambda g, j: (g, 0, j)),
            pl.BlockSpec((1, LK, W), lambda g, j: (g, 0, j)),
            pl.BlockSpec((1, LK, W), lambda g, j: (g, 0, j)),
            pl.BlockSpec((LQ, LK), lambda g, j: (0, 0)),
        ],
        out_specs=pl.BlockSpec((1, LQ, W), lambda g, j: (g, 0, j)),
        out_shape=jax.ShapeDtypeStruct((b, LQ, D_MODEL), jnp.float32),
        scratch_shapes=scratch,
    )(q, k, v, cnt)


def _conv_distil_krn(x_ref, w0_ref, w1_ref, w2_ref, b_ref, sc_ref, bb_ref,
                     o_ref, *, L):
    x = x_ref[0]
    xm1 = jnp.concatenate([x[L - 1:L, :], x[:L - 1, :]], axis=0)
    xp1 = jnp.concatenate([x[1:, :], x[:1, :]], axis=0)
    y = (jnp.dot(xm1, w0_ref[...], preferred_element_type=jnp.float32)
         + jnp.dot(x, w1_ref[...], preferred_element_type=jnp.float32)
         + jnp.dot(xp1, w2_ref[...], preferred_element_type=jnp.float32)
         + b_ref[...])
    y = y * sc_ref[...] + bb_ref[...]
    y = jnp.where(y > 0, y, jnp.exp(jnp.minimum(y, 0.0)) - 1.0)
    C = y.shape[1]
    neg = jnp.full((1, C), _NEG, jnp.float32)
    ym1 = jnp.concatenate([neg, y[:L - 1, :]], axis=0)
    yp1 = jnp.concatenate([y[1:, :], neg], axis=0)
    pf = jnp.maximum(jnp.maximum(ym1, y), yp1)
    r_io = jax.lax.broadcasted_iota(jnp.int32, (256, 512), 0)
    c_io = jax.lax.broadcasted_iota(jnp.int32, (256, 512), 1)
    dsel = (c_io == 2 * r_io).astype(jnp.float32)
    for t in range(L // 512):
        o_ref[0, t * 256:(t + 1) * 256, :] = jnp.dot(
            dsel, pf[t * 512:(t + 1) * 512, :],
            preferred_element_type=jnp.float32)


def _conv_distil(x, w0, w1, w2, b, scale, bb):
    b_, L, C = x.shape
    return pl.pallas_call(
        functools.partial(_conv_distil_krn, L=L),
        grid=(b_,),
        in_specs=[
            pl.BlockSpec((1, L, C), lambda i: (i, 0, 0)),
            pl.BlockSpec((C, C), lambda i: (0, 0)),
            pl.BlockSpec((C, C), lambda i: (0, 0)),
            pl.BlockSpec((C, C), lambda i: (0, 0)),
            pl.BlockSpec((1, C), lambda i: (0, 0)),
            pl.BlockSpec((1, C), lambda i: (0, 0)),
            pl.BlockSpec((1, C), lambda i: (0, 0)),
        ],
        out_specs=pl.BlockSpec((1, L // 2, C), lambda i: (i, 0, 0)),
        out_shape=jax.ShapeDtypeStruct((b_, L // 2, C), jnp.float32),
    )(x, w0, w1, w2, b, scale, bb)


def _final_krn(x_ref, g_ref, be_ref, w_ref, b_ref, o_ref):
    y = _ln(x_ref[...], g_ref[...], be_ref[...])
    o_ref[...] = jnp.dot(y, w_ref[...],
                         preferred_element_type=jnp.float32) + b_ref[...]


def _final_proj(x, g, beta, w, b):
    M, K = x.shape
    N = w.shape[1]
    TM = 512
    return pl.pallas_call(
        _final_krn,
        grid=(M // TM,),
        in_specs=[
            pl.BlockSpec((TM, K), lambda i: (i, 0)),
            pl.BlockSpec((1, K), lambda i: (0, 0)),
            pl.BlockSpec((1, K), lambda i: (0, 0)),
            pl.BlockSpec((K, N), lambda i: (0, 0)),
            pl.BlockSpec((1, N), lambda i: (0, 0)),
        ],
        out_specs=pl.BlockSpec((TM, N), lambda i: (i, 0)),
        out_shape=jax.ShapeDtypeStruct((M, N), jnp.float32),
    )(x, g, beta, w, b)


def _row(v):
    return v.reshape(1, -1)


def _build_xc(x, mark, conv_w, temp_w):
    xc = jnp.concatenate(
        [jnp.roll(x, 1, axis=1), x, jnp.roll(x, -1, axis=1), mark], axis=-1)
    kin = xc.shape[-1]
    pad = (-kin) % 32
    xc = jnp.pad(xc, ((0, 0), (0, 0), (0, pad)))
    w = jnp.concatenate([conv_w[:, :, 0].T, conv_w[:, :, 1].T,
                         conv_w[:, :, 2].T, temp_w.T], axis=0)
    w = jnp.pad(w, ((0, pad), (0, 0)))
    return xc, w


def _qkv_cat(p, names):
    w = jnp.concatenate([p['w' + n].T for n in names], axis=1)
    b = jnp.concatenate([p['b' + n] for n in names])
    return w, _row(b)


def _u_of(LQ):
    return min(FACTOR * int(np.ceil(np.log(LQ))), LQ)


def kernel(batch_x, batch_x_mark, batch_y_mark, params):
    p = params
    pos = jnp.asarray(_POS_2048)

    e0 = p['enc0']
    xc, we = _build_xc(batch_x, batch_x_mark,
                       p['enc_emb']['conv_w'], p['enc_emb']['temp_w'])
    wqkv, bqkv = _qkv_cat(e0['attn'], ['q', 'k', 'v'])
    emb, q, k, v = _embed_qkv(xc, we, pos, wqkv, bqkv)

    ctx = _fused_attn(q, k, v, jnp.asarray(_CNTS[0]), _u_of(SEQ), False)
    x2 = _proj_ffn(ctx.reshape(-1, D_MODEL), emb.reshape(-1, D_MODEL), e0['attn']['wo'].T,
                   _row(e0['attn']['bo']), _row(e0['ln1_g']), _row(e0['ln1_b']),
                   e0['w1'].T, _row(e0['b1']), e0['w2'].T, _row(e0['b2']),
                   _row(e0['ln2_g']), _row(e0['ln2_b']), B, SEQ)

    c0 = p['conv0']
    scale = _row(c0['bn_g'] * (1.0 / np.sqrt(1.0 + 1e-5)))
    x = _conv_distil(x2.reshape(B, SEQ, D_MODEL), c0['w'][:, :, 0].T,
                     c0['w'][:, :, 1].T, c0['w'][:, :, 2].T, _row(c0['b']),
                     scale, _row(c0['bn_b']))

    e1 = p['enc1']
    L1 = SEQ // 2
    wqkv, bqkv = _qkv_cat(e1['attn'], ['q', 'k', 'v'])
    q, k, v = _qkv_proj(x, wqkv, bqkv, 3)
    ctx = _fused_attn(q, k, v, jnp.asarray(_CNTS[1]), _u_of(L1), False)
    x2 = _proj_ffn(ctx.reshape(-1, D_MODEL), x.reshape(-1, D_MODEL), e1['attn']['wo'].T,
                   _row(e1['attn']['bo']), _row(e1['ln1_g']), _row(e1['ln1_b']),
                   e1['w1'].T, _row(e1['b1']), e1['w2'].T, _row(e1['b2']),
                   _row(e1['ln2_g']), _row(e1['ln2_b']), B, L1,
                   _row(p['enc_norm_g']), _row(p['enc_norm_b']))
    enc_out = x2.reshape(B, L1, D_MODEL)

    d0 = p['dec0']
    L_DEC = LABEL_LEN + PRED_LEN
    dec_inp = jnp.concatenate(
        [batch_x[:, -LABEL_LEN:, :],
         jnp.zeros((B, PRED_LEN, ENC_IN), jnp.float32)], axis=1)
    xc, we = _build_xc(dec_inp, batch_y_mark,
                       p['dec_emb']['conv_w'], p['dec_emb']['temp_w'])
    wqkv, bqkv = _qkv_cat(d0['self'], ['q', 'k', 'v'])
    emb, q, k, v = _embed_qkv(xc, we, pos, wqkv, bqkv)

    ctx = _fused_attn(q, k, v, jnp.asarray(_CNTS[2]), _u_of(L_DEC), True)
    t = _proj_res_ln(ctx.reshape(-1, D_MODEL), emb.reshape(-1, D_MODEL), d0['self']['wo'].T,
                     _row(d0['self']['bo']), _row(d0['ln1_g']),
                     _row(d0['ln1_b']), B, L_DEC)

    q2, = _qkv_proj(t.reshape(B, L_DEC, D_MODEL), d0['cross']['wq'].T,
                    _row(d0['cross']['bq']), 1)
    wkv, bkv = _qkv_cat(d0['cross'], ['k', 'v'])
    k2, v2 = _qkv_proj(enc_out, wkv, bkv, 2)
    ctx2 = _fused_attn(q2, k2, v2, jnp.asarray(_CNTS[3]), _u_of(L_DEC), False)
    x2 = _proj_ffn(ctx2.reshape(-1, D_MODEL), t, d0['cross']['wo'].T, _row(d0['cross']['bo']),
                   _row(d0['ln2_g']), _row(d0['ln2_b']),
                   d0['w1'].T, _row(d0['b1']), d0['w2'].T, _row(d0['b2']),
                   _row(d0['ln3_g']), _row(d0['ln3_b']), B, L_DEC)

    xdec = x2.reshape(B, L_DEC, D_MODEL)[:, -PRED_LEN:, :]
    out = _final_proj(xdec.reshape(B * PRED_LEN, D_MODEL),
                      _row(p['dec_norm_g']), _row(p['dec_norm_b']),
                      p['proj_w'].T, _row(p['proj_b']))
    return out.reshape(B, PRED_LEN, C_OUT)
